# Initial kernel scaffold; baseline (speedup 1.0000x reference)
import jax
import jax.numpy as jnp
from jax.experimental import pallas as pl


def kernel(edge_tokens, node_tokens, question_tokens, ln_g, ln_b, W1, b1, W2, b2, edge_batch, selected_mask, edge_index):
    raise NotImplementedError("write your pallas kernel here")


# speedup vs baseline: 7.1365x; 7.1365x over previous
import functools

import jax
import jax.numpy as jnp
from jax import lax
from jax.experimental import pallas as pl
from jax.experimental.pallas import tpu as pltpu
from jax.experimental.pallas import tpu_sc as plsc

_N_NODES = 10000
_N_EDGES = 320000
_HID = 128
_NG = 16

_NW = 32
_EPT = _N_EDGES // _NW
_CH = 40
_NCH = _EPT // _CH
_NPAD = 10240
_EB = 2560
_NB = _N_EDGES // _EB


def _sc_mesh():
    return plsc.VectorSubcoreMesh(
        core_axis_name="c", subcore_axis_name="s", num_cores=2, num_subcores=16
    )


def _wid():
    return lax.axis_index("s") * 2 + lax.axis_index("c")


def _sc_gather_tails(node_tokens, tgt3):
    @functools.partial(
        pl.kernel,
        mesh=_sc_mesh(),
        out_type=jax.ShapeDtypeStruct((_N_EDGES, _HID), jnp.float32),
        scratch_types=[
            pltpu.VMEM((_NCH, _CH), jnp.int32),
            pltpu.VMEM((_CH, _HID), jnp.float32),
            pltpu.VMEM((_CH, _HID), jnp.float32),
            pltpu.SemaphoreType.DMA,
            pltpu.SemaphoreType.DMA,
        ],
    )
    def k(node_hbm, tgt_hbm, out_hbm, idx_v, buf0, buf1, sem0, sem1):
        wid = _wid()
        base = wid * _EPT
        pltpu.sync_copy(tgt_hbm.at[wid], idx_v)
        pltpu.async_copy(node_hbm.at[idx_v.at[0]], buf0, sem0)
        pltpu.async_copy(node_hbm.at[idx_v.at[1]], buf1, sem1)

        def step(c, buf, sem):
            pltpu.make_async_copy(node_hbm.at[idx_v.at[c]], buf, sem).wait()
            pltpu.sync_copy(buf, out_hbm.at[pl.ds(base + c * _CH, _CH)])

            @pl.when(c + 2 < _NCH)
            def _():
                pltpu.async_copy(node_hbm.at[idx_v.at[c + 2]], buf, sem)

        def body(c, carry):
            @pl.when(c % 2 == 0)
            def _():
                step(c, buf0, sem0)

            @pl.when(c % 2 == 1)
            def _():
                step(c, buf1, sem1)

            return carry

        lax.fori_loop(0, _NCH, body, 0)

    return k(node_tokens, tgt3)


def _tc_logits(edge_tokens, tails, eb_col, q, lng_col, lnb_row, W1, b1_row,
               W2, b2_2d):
    def body(et_ref, tl_ref, eb_ref, q_ref, lng_ref, lnb_ref, w1_ref, b1_ref,
             w2_ref, b2_ref, out_ref, mx_ref):
        f32 = jnp.float32
        e = et_ref[...]
        t = tl_ref[...]
        w1 = w1_ref[...]
        w1g = w1 * lng_ref[...]
        wbar = jnp.sum(w1g, axis=0, keepdims=True)
        hb = jnp.dot(lnb_ref[...], w1, preferred_element_type=f32) + b1_ref[...]
        w1e = w1g[:_HID]
        w1q = w1g[_HID:2 * _HID]
        w1t = w1g[2 * _HID:]
        qt = q_ref[...]
        qp = jnp.dot(qt, w1q, preferred_element_type=f32)
        sq = jnp.sum(qt, axis=1, keepdims=True)
        sqq = jnp.sum(qt * qt, axis=1, keepdims=True)
        gids = lax.broadcasted_iota(jnp.int32, (1, _NG), 1)
        onehot = (eb_ref[...] == gids).astype(f32)
        qpe = jnp.dot(onehot, qp, preferred_element_type=f32)
        sqe = jnp.dot(onehot, sq, preferred_element_type=f32)
        sqqe = jnp.dot(onehot, sqq, preferred_element_type=f32)
        xw = (jnp.dot(e, w1e, preferred_element_type=f32)
              + jnp.dot(t, w1t, preferred_element_type=f32) + qpe)
        se = jnp.sum(e, axis=1, keepdims=True)
        sse = jnp.sum(e * e, axis=1, keepdims=True)
        st = jnp.sum(t, axis=1, keepdims=True)
        sst = jnp.sum(t * t, axis=1, keepdims=True)
        mu = (se + st + sqe) * (1.0 / 384.0)
        var = (sse + sst + sqqe) * (1.0 / 384.0) - mu * mu
        r = lax.rsqrt(var + 1e-5)
        h = xw * r - (mu * r) * wbar + hb
        gel = 0.5 * h * (1.0 + lax.erf(h * 0.7071067811865476))
        lg = jnp.dot(gel, w2_ref[...], preferred_element_type=f32) \
            + b2_ref[...]
        out_ref[...] = lg
        m = jnp.full((1, 16), jnp.max(lg), f32)

        @pl.when(pl.program_id(0) == 0)
        def _():
            mx_ref[...] = m

        @pl.when(pl.program_id(0) > 0)
        def _():
            mx_ref[...] = jnp.maximum(mx_ref[...], m)

    return pl.pallas_call(
        body,
        grid=(_NB,),
        in_specs=[
            pl.BlockSpec((_EB, _HID), lambda i: (i, 0)),
            pl.BlockSpec((_EB, _HID), lambda i: (i, 0)),
            pl.BlockSpec((_EB, 1), lambda i: (i, 0)),
            pl.BlockSpec((_NG, _HID), lambda i: (0, 0)),
            pl.BlockSpec((3 * _HID, 1), lambda i: (0, 0)),
            pl.BlockSpec((1, 3 * _HID), lambda i: (0, 0)),
            pl.BlockSpec((3 * _HID, _HID), lambda i: (0, 0)),
            pl.BlockSpec((1, _HID), lambda i: (0, 0)),
            pl.BlockSpec((_HID, 1), lambda i: (0, 0)),
            pl.BlockSpec((1, 1), lambda i: (0, 0)),
        ],
        out_specs=[
            pl.BlockSpec((_EB, 1), lambda i: (i, 0)),
            pl.BlockSpec((1, 16), lambda i: (0, 0)),
        ],
        out_shape=[
            jax.ShapeDtypeStruct((_N_EDGES, 1), jnp.float32),
            jax.ShapeDtypeStruct((1, 16), jnp.float32),
        ],
    )(edge_tokens, tails, eb_col, q, lng_col, lnb_row, W1, b1_row, W2, b2_2d)


_CHS = 80
_NCHS = _EPT // _CHS


def _sc_denom(lg2, tgt3, gmax16):
    @functools.partial(
        pl.kernel,
        mesh=_sc_mesh(),
        out_type=jax.ShapeDtypeStruct((2, _NPAD), jnp.float32),
        scratch_types=[
            pltpu.VMEM((_EPT,), jnp.float32),
            pltpu.VMEM((_NCHS, _CHS), jnp.int32),
            pltpu.VMEM((16,), jnp.float32),
            pltpu.VMEM((_NPAD,), jnp.float32),
            pltpu.VMEM_SHARED((_NPAD,), jnp.float32),
        ],
    )
    def k(lg_hbm, tgt_hbm, gm_hbm, out_hbm, lg_v, ix_v, gm_v, z_v, acc_sh):
        wid = _wid()
        core = lax.axis_index("c")
        sid = lax.axis_index("s")
        pltpu.sync_copy(lg_hbm.at[wid], lg_v)
        pltpu.sync_copy(tgt_hbm.at[wid], ix_v)
        pltpu.sync_copy(gm_hbm, gm_v)
        m = gm_v[...]

        @pl.when(sid == 0)
        def _():
            zero = jnp.zeros((16,), jnp.float32)

            def zb(i, c):
                z_v[pl.ds(i * 16, 16)] = zero
                return c

            lax.fori_loop(0, _NPAD // 16, zb, 0)
            pltpu.sync_copy(z_v, acc_sh)

        def eb(j, c):
            v = lg_v[pl.ds(j * 16, 16)]
            lg_v[pl.ds(j * 16, 16)] = jnp.exp(v - m)
            return c

        lax.fori_loop(0, _EPT // 16, eb, 0)
        plsc.subcore_barrier()

        def sb(ci, c):
            pltpu.sync_copy(
                lg_v.at[pl.ds(ci * _CHS, _CHS)],
                acc_sh.at[ix_v.at[ci]],
                add=True,
            )
            return c

        lax.fori_loop(0, _NCHS, sb, 0)
        plsc.subcore_barrier()

        @pl.when(sid == 0)
        def _():
            pltpu.sync_copy(acc_sh, out_hbm.at[core])

    return k(lg2, tgt3, gmax16)


def _tc_nodeoff(parts, gmax):
    def body(p_ref, m_ref, o_ref):
        den = jnp.sum(p_ref[...], axis=0, keepdims=True)
        o_ref[...] = jnp.max(m_ref[...]) + jnp.log(den)

    return pl.pallas_call(
        body, out_shape=jax.ShapeDtypeStruct((1, _NPAD), jnp.float32)
    )(parts, gmax)


def _sc_gatheroff(nodeoff, tgt3):
    @functools.partial(
        pl.kernel,
        mesh=_sc_mesh(),
        out_type=jax.ShapeDtypeStruct((_N_EDGES,), jnp.float32),
        scratch_types=[
            pltpu.VMEM((_NCHS, _CHS), jnp.int32),
            pltpu.VMEM_SHARED((_NPAD,), jnp.float32),
            pltpu.VMEM((_EPT,), jnp.float32),
        ],
    )
    def k(off_hbm, tgt_hbm, out_hbm, idx_v, off_sh, res_v):
        wid = _wid()
        sid = lax.axis_index("s")
        pltpu.sync_copy(tgt_hbm.at[wid], idx_v)

        @pl.when(sid == 0)
        def _():
            pltpu.sync_copy(off_hbm, off_sh)

        plsc.subcore_barrier()

        def body(ci, c):
            pltpu.sync_copy(
                off_sh.at[idx_v.at[ci]], res_v.at[pl.ds(ci * _CHS, _CHS)]
            )
            return c

        lax.fori_loop(0, _NCHS, body, 0)
        pltpu.sync_copy(res_v, out_hbm.at[pl.ds(wid * _EPT, _EPT)])

    return k(nodeoff, tgt3)


def _tc_final(logits, offsel, eb_col, sel_col):
    def body(lg_ref, off_ref, eb_ref, sl_ref, o1_ref, o2_ref, o3_ref):
        f32 = jnp.float32
        i = pl.program_id(0)
        gids = lax.broadcasted_iota(jnp.int32, (1, _NG), 1)
        onehot = (eb_ref[...] == gids).astype(f32)
        s = sl_ref[...].astype(f32)
        lps = (lg_ref[...] - off_ref[...]) * s
        part = jnp.sum(onehot * lps, axis=0, keepdims=True)
        pcnt = jnp.sum(onehot * s, axis=0, keepdims=True)

        @pl.when(i == 0)
        def _():
            o1_ref[...] = part
            o2_ref[...] = pcnt

        @pl.when(i > 0)
        def _():
            o1_ref[...] = o1_ref[...] + part
            o2_ref[...] = o2_ref[...] + pcnt

        @pl.when(i == _NB - 1)
        def _():
            tot = o1_ref[...]
            cnt = o2_ref[...]
            has = cnt > 0.0
            nll = jnp.where(has, -tot, 0.0)
            ngr = jnp.maximum(jnp.sum(has.astype(f32)), 1.0)
            o3_ref[...] = (jnp.sum(nll) / ngr).reshape(1, 1)

    return pl.pallas_call(
        body,
        grid=(_NB,),
        in_specs=[
            pl.BlockSpec((_EB, 1), lambda i: (i, 0)),
            pl.BlockSpec((_EB, 1), lambda i: (i, 0)),
            pl.BlockSpec((_EB, 1), lambda i: (i, 0)),
            pl.BlockSpec((_EB, 1), lambda i: (i, 0)),
        ],
        out_specs=[
            pl.BlockSpec((1, _NG), lambda i: (0, 0)),
            pl.BlockSpec((1, _NG), lambda i: (0, 0)),
            pl.BlockSpec((1, 1), lambda i: (0, 0)),
        ],
        out_shape=[
            jax.ShapeDtypeStruct((1, _NG), jnp.float32),
            jax.ShapeDtypeStruct((1, _NG), jnp.float32),
            jax.ShapeDtypeStruct((1, 1), jnp.float32),
        ],
    )(logits, offsel, eb_col, sel_col)


def kernel(edge_tokens, node_tokens, question_tokens, ln_g, ln_b, W1, b1, W2,
           b2, edge_batch, selected_mask, edge_index):
    tgt = edge_index[1]
    tails = _sc_gather_tails(node_tokens, tgt.reshape(_NW, _NCH, _CH))
    logits, gmax = _tc_logits(
        edge_tokens, tails,
        edge_batch.reshape(_N_EDGES, 1),
        question_tokens,
        ln_g.reshape(3 * _HID, 1),
        ln_b.reshape(1, 3 * _HID),
        W1,
        b1.reshape(1, _HID),
        W2,
        b2.reshape(1, 1),
    )
    lg2 = logits.reshape(_NW, _EPT)
    tgt2 = tgt.reshape(_NW, _EPT)
    denp = _sc_denom(lg2, tgt.reshape(_NW, _NCHS, _CHS), gmax.reshape(16))
    nodeoff = _tc_nodeoff(denp, gmax)
    offsel = _sc_gatheroff(
        nodeoff.reshape(_NPAD), tgt.reshape(_NW, _NCHS, _CHS)
    )
    logpb, _, nll = _tc_final(
        logits,
        offsel.reshape(_N_EDGES, 1),
        edge_batch.reshape(_N_EDGES, 1),
        selected_mask.astype(jnp.int32).reshape(_N_EDGES, 1),
    )
    return logpb.reshape(_NG), nll.reshape(())


# speedup vs baseline: 12.1300x; 1.6997x over previous
import functools

import jax
import jax.numpy as jnp
from jax import lax
from jax.experimental import pallas as pl
from jax.experimental.pallas import tpu as pltpu
from jax.experimental.pallas import tpu_sc as plsc

_N_NODES = 10000
_N_EDGES = 320000
_HID = 128
_NG = 16

_NW = 32
_EPT = _N_EDGES // _NW
_CH = 40
_NCH = _EPT // _CH
_NPAD = 10240
_EB = 2560
_NB = _N_EDGES // _EB


def _sc_mesh():
    return plsc.VectorSubcoreMesh(
        core_axis_name="c", subcore_axis_name="s", num_cores=2, num_subcores=16
    )


def _wid():
    return lax.axis_index("s") * 2 + lax.axis_index("c")


def _sc_gather_tails(node_tokens, tgt3):
    @functools.partial(
        pl.kernel,
        mesh=_sc_mesh(),
        out_type=jax.ShapeDtypeStruct((_N_EDGES, _HID), jnp.float32),
        scratch_types=[
            pltpu.VMEM((_NCH, _CH), jnp.int32),
            pltpu.VMEM((_CH, _HID), jnp.float32),
            pltpu.VMEM((_CH, _HID), jnp.float32),
            pltpu.SemaphoreType.DMA,
            pltpu.SemaphoreType.DMA,
        ],
    )
    def k(node_hbm, tgt_hbm, out_hbm, idx_v, buf0, buf1, sem0, sem1):
        wid = _wid()
        base = wid * _EPT
        pltpu.sync_copy(tgt_hbm.at[wid], idx_v)
        pltpu.async_copy(node_hbm.at[idx_v.at[0]], buf0, sem0)
        pltpu.async_copy(node_hbm.at[idx_v.at[1]], buf1, sem1)

        def step(c, buf, sem):
            pltpu.make_async_copy(node_hbm.at[idx_v.at[c]], buf, sem).wait()
            pltpu.sync_copy(buf, out_hbm.at[pl.ds(base + c * _CH, _CH)])

            @pl.when(c + 2 < _NCH)
            def _():
                pltpu.async_copy(node_hbm.at[idx_v.at[c + 2]], buf, sem)

        def body(c, carry):
            @pl.when(c % 2 == 0)
            def _():
                step(c, buf0, sem0)

            @pl.when(c % 2 == 1)
            def _():
                step(c, buf1, sem1)

            return carry

        lax.fori_loop(0, _NCH, body, 0)

    return k(node_tokens, tgt3)


def _tc_logits(edge_tokens, tails, st_row, en_row, q, lng_col, lnb_row, W1,
               b1_row, W2, b2_2d):
    def body(et_ref, tl_ref, st_ref, en_ref, q_ref, lng_ref, lnb_ref, b1_ref,
             w1_ref, w2_ref, b2_ref, out_ref, mx_ref):
        f32 = jnp.float32
        e = et_ref[...]
        t = tl_ref[...]
        w1 = w1_ref[...]
        w1g = w1 * lng_ref[...]
        wbar = jnp.sum(w1g, axis=0, keepdims=True)
        hb = jnp.dot(lnb_ref[...], w1, preferred_element_type=f32) + b1_ref[...]
        w1e = w1g[:_HID]
        w1q = w1g[_HID:2 * _HID]
        w1t = w1g[2 * _HID:]
        qt = q_ref[...]
        qp = jnp.dot(qt, w1q, preferred_element_type=f32)
        sq = jnp.sum(qt, axis=1, keepdims=True)
        sqq = jnp.sum(qt * qt, axis=1, keepdims=True)
        eid = pl.program_id(0) * _EB + lax.broadcasted_iota(
            jnp.int32, (_EB, 1), 0
        )
        onehot = ((eid >= st_ref[...]) & (eid < en_ref[...])).astype(f32)
        qpe = jnp.dot(onehot, qp, preferred_element_type=f32)
        sqe = jnp.dot(onehot, sq, preferred_element_type=f32)
        sqqe = jnp.dot(onehot, sqq, preferred_element_type=f32)
        xw = (jnp.dot(e, w1e, preferred_element_type=f32)
              + jnp.dot(t, w1t, preferred_element_type=f32) + qpe)
        se = jnp.sum(e, axis=1, keepdims=True)
        sse = jnp.sum(e * e, axis=1, keepdims=True)
        st = jnp.sum(t, axis=1, keepdims=True)
        sst = jnp.sum(t * t, axis=1, keepdims=True)
        mu = (se + st + sqe) * (1.0 / 384.0)
        var = (sse + sst + sqqe) * (1.0 / 384.0) - mu * mu
        r = lax.rsqrt(var + 1e-5)
        h = xw * r - (mu * r) * wbar + hb
        gel = 0.5 * h * (1.0 + lax.erf(h * 0.7071067811865476))
        lg = jnp.dot(gel, w2_ref[...], preferred_element_type=f32) \
            + b2_ref[...]
        out_ref[...] = lg
        m = jnp.full((1, 16), jnp.max(lg), f32)

        @pl.when(pl.program_id(0) == 0)
        def _():
            mx_ref[...] = m

        @pl.when(pl.program_id(0) > 0)
        def _():
            mx_ref[...] = jnp.maximum(mx_ref[...], m)

    return pl.pallas_call(
        body,
        grid=(_NB,),
        in_specs=[
            pl.BlockSpec((_EB, _HID), lambda i: (i, 0)),
            pl.BlockSpec((_EB, _HID), lambda i: (i, 0)),
            pl.BlockSpec((1, _NG), lambda i: (0, 0)),
            pl.BlockSpec((1, _NG), lambda i: (0, 0)),
            pl.BlockSpec((_NG, _HID), lambda i: (0, 0)),
            pl.BlockSpec((3 * _HID, 1), lambda i: (0, 0)),
            pl.BlockSpec((1, 3 * _HID), lambda i: (0, 0)),
            pl.BlockSpec((1, _HID), lambda i: (0, 0)),
            pl.BlockSpec((3 * _HID, _HID), lambda i: (0, 0)),
            pl.BlockSpec((_HID, 1), lambda i: (0, 0)),
            pl.BlockSpec((1, 1), lambda i: (0, 0)),
        ],
        out_specs=[
            pl.BlockSpec((_EB, 1), lambda i: (i, 0)),
            pl.BlockSpec((1, 16), lambda i: (0, 0)),
        ],
        out_shape=[
            jax.ShapeDtypeStruct((_N_EDGES, 1), jnp.float32),
            jax.ShapeDtypeStruct((1, 16), jnp.float32),
        ],
    )(edge_tokens, tails, st_row, en_row, q, lng_col, lnb_row, b1_row, W1,
      W2, b2_2d)


_CHS = 80
_NCHS = _EPT // _CHS


def _sc_denom(lg2, tgt3, gmax16):
    @functools.partial(
        pl.kernel,
        mesh=_sc_mesh(),
        out_type=jax.ShapeDtypeStruct((2, _NPAD), jnp.float32),
        scratch_types=[
            pltpu.VMEM((_EPT,), jnp.float32),
            pltpu.VMEM((_NCHS, _CHS), jnp.int32),
            pltpu.VMEM((16,), jnp.float32),
            pltpu.VMEM((_NPAD,), jnp.float32),
            pltpu.VMEM_SHARED((_NPAD,), jnp.float32),
        ],
    )
    def k(lg_hbm, tgt_hbm, gm_hbm, out_hbm, lg_v, ix_v, gm_v, z_v, acc_sh):
        wid = _wid()
        core = lax.axis_index("c")
        sid = lax.axis_index("s")
        pltpu.sync_copy(lg_hbm.at[wid], lg_v)
        pltpu.sync_copy(tgt_hbm.at[wid], ix_v)
        pltpu.sync_copy(gm_hbm, gm_v)
        m = gm_v[...]

        @pl.when(sid == 0)
        def _():
            zero = jnp.zeros((16,), jnp.float32)

            def zb(i, c):
                z_v[pl.ds(i * 16, 16)] = zero
                return c

            lax.fori_loop(0, _NPAD // 16, zb, 0)
            pltpu.sync_copy(z_v, acc_sh)

        def eb(j, c):
            v = lg_v[pl.ds(j * 16, 16)]
            lg_v[pl.ds(j * 16, 16)] = jnp.exp(v - m)
            return c

        lax.fori_loop(0, _EPT // 16, eb, 0)
        plsc.subcore_barrier()

        def sb(ci, c):
            pltpu.sync_copy(
                lg_v.at[pl.ds(ci * _CHS, _CHS)],
                acc_sh.at[ix_v.at[ci]],
                add=True,
            )
            return c

        lax.fori_loop(0, _NCHS, sb, 0)
        plsc.subcore_barrier()

        @pl.when(sid == 0)
        def _():
            pltpu.sync_copy(acc_sh, out_hbm.at[core])

    return k(lg2, tgt3, gmax16)


def _tc_nodeoff(parts, gmax):
    def body(p_ref, m_ref, o_ref):
        den = jnp.sum(p_ref[...], axis=0, keepdims=True)
        o_ref[...] = jnp.max(m_ref[...]) + jnp.log(den)

    return pl.pallas_call(
        body, out_shape=jax.ShapeDtypeStruct((1, _NPAD), jnp.float32)
    )(parts, gmax)


def _sc_graphsum(nodeoff, tgt3, lg2, eb3, sel3):
    @functools.partial(
        pl.kernel,
        mesh=_sc_mesh(),
        out_type=jax.ShapeDtypeStruct((2, 512), jnp.float32),
        scratch_types=[
            pltpu.VMEM((_NCHS, _CHS), jnp.int32),
            pltpu.VMEM_SHARED((_NPAD,), jnp.float32),
            pltpu.VMEM((_EPT,), jnp.float32),
            pltpu.VMEM((_EPT,), jnp.float32),
            pltpu.VMEM((_NCHS, _CHS), jnp.float32),
            pltpu.VMEM((_NCHS, _CHS), jnp.float32),
            pltpu.VMEM((_NCHS, _CHS), jnp.int32),
            pltpu.VMEM((_NCHS, _CHS), jnp.int32),
            pltpu.VMEM((512,), jnp.float32),
            pltpu.VMEM_SHARED((512,), jnp.float32),
        ],
    )
    def k(off_hbm, tgt_hbm, lg_hbm, eb_hbm, sel_hbm, out_hbm,
          idx_v, off_sh, ofs_v, lg_v, lps_v, sel_v, kv_v, kv2_v, z_v, acc_sh):
        wid = _wid()
        core = lax.axis_index("c")
        sid = lax.axis_index("s")
        pltpu.sync_copy(tgt_hbm.at[wid], idx_v)
        pltpu.sync_copy(lg_hbm.at[wid], lg_v)
        pltpu.sync_copy(eb_hbm.at[wid], kv_v)
        pltpu.sync_copy(sel_hbm.at[wid], sel_v)

        @pl.when(sid == 0)
        def _():
            pltpu.sync_copy(off_hbm, off_sh)
            zero = jnp.zeros((16,), jnp.float32)

            def zb(i, c):
                z_v[pl.ds(i * 16, 16)] = zero
                return c

            lax.fori_loop(0, 512 // 16, zb, 0)
            pltpu.sync_copy(z_v, acc_sh)

        plsc.subcore_barrier()

        def gb(ci, c):
            pltpu.sync_copy(
                off_sh.at[idx_v.at[ci]], ofs_v.at[pl.ds(ci * _CHS, _CHS)]
            )
            return c

        lax.fori_loop(0, _NCHS, gb, 0)

        kbase = sid * 16

        def cb(j, c):
            ci = j // (_CHS // 16)
            l = (j % (_CHS // 16)) * 16
            v = lg_v[pl.ds(j * 16, 16)]
            o = ofs_v[pl.ds(j * 16, 16)]
            s = sel_v[ci, pl.ds(l, 16)]
            g = kv_v[ci, pl.ds(l, 16)]
            lps_v[ci, pl.ds(l, 16)] = (v - o) * s
            kv_v[ci, pl.ds(l, 16)] = g + kbase
            kv2_v[ci, pl.ds(l, 16)] = g + (kbase + 256)
            return c

        lax.fori_loop(0, _EPT // 16, cb, 0)

        def sb(ci, c):
            pltpu.sync_copy(lps_v.at[ci], acc_sh.at[kv_v.at[ci]], add=True)
            pltpu.sync_copy(sel_v.at[ci], acc_sh.at[kv2_v.at[ci]], add=True)
            return c

        lax.fori_loop(0, _NCHS, sb, 0)
        plsc.subcore_barrier()

        @pl.when(sid == 0)
        def _():
            pltpu.sync_copy(acc_sh, out_hbm.at[core])

    return k(nodeoff, tgt3, lg2, eb3, sel3)


def _tc_final(parts):
    def body(p_ref, o1_ref, o2_ref):
        f32 = jnp.float32
        x = p_ref[...]
        tot = (jnp.sum(x[0:16], axis=0, keepdims=True)
               + jnp.sum(x[32:48], axis=0, keepdims=True))
        cnt = (jnp.sum(x[16:32], axis=0, keepdims=True)
               + jnp.sum(x[48:64], axis=0, keepdims=True))
        has = cnt > 0.0
        nll = jnp.where(has, -tot, 0.0)
        ngr = jnp.maximum(jnp.sum(has.astype(f32)), 1.0)
        o1_ref[...] = tot
        o2_ref[...] = (jnp.sum(nll) / ngr).reshape(1, 1)

    return pl.pallas_call(
        body,
        out_shape=[
            jax.ShapeDtypeStruct((1, _NG), jnp.float32),
            jax.ShapeDtypeStruct((1, 1), jnp.float32),
        ],
    )(parts)


def kernel(edge_tokens, node_tokens, question_tokens, ln_g, ln_b, W1, b1, W2,
           b2, edge_batch, selected_mask, edge_index):
    tgt = edge_index[1]
    tails = _sc_gather_tails(node_tokens, tgt.reshape(_NW, _NCH, _CH))
    starts = jnp.searchsorted(
        edge_batch, jnp.arange(_NG, dtype=edge_batch.dtype)
    ).astype(jnp.int32)
    ends = jnp.concatenate(
        [starts[1:], jnp.full((1,), _N_EDGES, jnp.int32)]
    )
    logits, gmax = _tc_logits(
        edge_tokens, tails,
        starts.reshape(1, _NG),
        ends.reshape(1, _NG),
        question_tokens,
        ln_g.reshape(3 * _HID, 1),
        ln_b.reshape(1, 3 * _HID),
        W1,
        b1.reshape(1, _HID),
        W2,
        b2.reshape(1, 1),
    )
    lg2 = logits.reshape(_NW, _EPT)
    tgt3 = tgt.reshape(_NW, _NCHS, _CHS)
    denp = _sc_denom(lg2, tgt3, gmax.reshape(16))
    nodeoff = _tc_nodeoff(denp, gmax)
    parts = _sc_graphsum(
        nodeoff.reshape(_NPAD), tgt3, lg2,
        edge_batch.reshape(_NW, _NCHS, _CHS),
        selected_mask.astype(jnp.float32).reshape(_NW, _NCHS, _CHS),
    )
    logpb, nll = _tc_final(parts.reshape(64, _NG))
    return logpb.reshape(_NG), nll.reshape(())


# speedup vs baseline: 12.3035x; 1.0143x over previous
import functools

import jax
import jax.numpy as jnp
from jax import lax
from jax.experimental import pallas as pl
from jax.experimental.pallas import tpu as pltpu
from jax.experimental.pallas import tpu_sc as plsc

_N_NODES = 10000
_N_EDGES = 320000
_HID = 128
_NG = 16

_NW = 32
_EPT = _N_EDGES // _NW
_CH = 40
_NCH = _EPT // _CH
_NPAD = 10240
_EB = 2560
_NB = _N_EDGES // _EB


def _sc_mesh():
    return plsc.VectorSubcoreMesh(
        core_axis_name="c", subcore_axis_name="s", num_cores=2, num_subcores=16
    )


def _wid():
    return lax.axis_index("s") * 2 + lax.axis_index("c")


def _sc_gather_tails(node_tokens, tgt3):
    @functools.partial(
        pl.kernel,
        mesh=_sc_mesh(),
        out_type=jax.ShapeDtypeStruct((_N_EDGES, _HID), jnp.float32),
        scratch_types=[
            pltpu.VMEM((_NCH, _CH), jnp.int32),
            pltpu.VMEM((_CH, _HID), jnp.float32),
            pltpu.VMEM((_CH, _HID), jnp.float32),
            pltpu.SemaphoreType.DMA,
            pltpu.SemaphoreType.DMA,
        ],
    )
    def k(node_hbm, tgt_hbm, out_hbm, idx_v, buf0, buf1, sem0, sem1):
        wid = _wid()
        base = wid * _EPT
        pltpu.sync_copy(tgt_hbm.at[wid], idx_v)
        pltpu.async_copy(node_hbm.at[idx_v.at[0]], buf0, sem0)
        pltpu.async_copy(node_hbm.at[idx_v.at[1]], buf1, sem1)

        def step(c, buf, sem):
            pltpu.make_async_copy(node_hbm.at[idx_v.at[c]], buf, sem).wait()
            pltpu.sync_copy(buf, out_hbm.at[pl.ds(base + c * _CH, _CH)])

            @pl.when(c + 2 < _NCH)
            def _():
                pltpu.async_copy(node_hbm.at[idx_v.at[c + 2]], buf, sem)

        def body(c, carry):
            @pl.when(c % 2 == 0)
            def _():
                step(c, buf0, sem0)

            @pl.when(c % 2 == 1)
            def _():
                step(c, buf1, sem1)

            return carry

        lax.fori_loop(0, _NCH, body, 0)

    return k(node_tokens, tgt3)


def _tc_logits(edge_tokens, tails, st_row, en_row, q, lng_col, lnb_row, W1,
               b1_row, W2, b2_2d):
    def body(et_ref, tl_ref, st_ref, en_ref, q_ref, lng_ref, lnb_ref, b1_ref,
             w1_ref, w2_ref, b2_ref, out_ref, mx_ref):
        f32 = jnp.float32
        e = et_ref[...]
        t = tl_ref[...]
        w1 = w1_ref[...]
        w1g = w1 * lng_ref[...]
        wbar = jnp.sum(w1g, axis=0, keepdims=True)
        hb = jnp.dot(lnb_ref[...], w1, preferred_element_type=f32) + b1_ref[...]
        w1e = w1g[:_HID]
        w1q = w1g[_HID:2 * _HID]
        w1t = w1g[2 * _HID:]
        qt = q_ref[...]
        qp = jnp.dot(qt, w1q, preferred_element_type=f32)
        sq = jnp.sum(qt, axis=1, keepdims=True)
        sqq = jnp.sum(qt * qt, axis=1, keepdims=True)
        eid = pl.program_id(0) * _EB + lax.broadcasted_iota(
            jnp.int32, (_EB, 1), 0
        )
        onehot = ((eid >= st_ref[...]) & (eid < en_ref[...])).astype(f32)
        qpe = jnp.dot(onehot, qp, preferred_element_type=f32)
        sqe = jnp.dot(onehot, sq, preferred_element_type=f32)
        sqqe = jnp.dot(onehot, sqq, preferred_element_type=f32)
        xw = (jnp.dot(e, w1e, preferred_element_type=f32)
              + jnp.dot(t, w1t, preferred_element_type=f32) + qpe)
        ones_col = jnp.ones((_HID, 1), f32)
        u = e + t
        v = e * e + t * t
        suv = jnp.dot(u, ones_col, preferred_element_type=f32)
        svv = jnp.dot(v, ones_col, preferred_element_type=f32)
        mu = (suv + sqe) * (1.0 / 384.0)
        var = (svv + sqqe) * (1.0 / 384.0) - mu * mu
        r = lax.rsqrt(var + 1e-5)
        h = xw * r - (mu * r) * wbar + hb
        gel = 0.5 * h * (1.0 + lax.erf(h * 0.7071067811865476))
        lg = jnp.dot(gel, w2_ref[...], preferred_element_type=f32) \
            + b2_ref[...]
        out_ref[...] = lg
        m = jnp.full((1, 16), jnp.max(lg), f32)

        @pl.when(pl.program_id(0) == 0)
        def _():
            mx_ref[...] = m

        @pl.when(pl.program_id(0) > 0)
        def _():
            mx_ref[...] = jnp.maximum(mx_ref[...], m)

    return pl.pallas_call(
        body,
        grid=(_NB,),
        in_specs=[
            pl.BlockSpec((_EB, _HID), lambda i: (i, 0)),
            pl.BlockSpec((_EB, _HID), lambda i: (i, 0)),
            pl.BlockSpec((1, _NG), lambda i: (0, 0)),
            pl.BlockSpec((1, _NG), lambda i: (0, 0)),
            pl.BlockSpec((_NG, _HID), lambda i: (0, 0)),
            pl.BlockSpec((3 * _HID, 1), lambda i: (0, 0)),
            pl.BlockSpec((1, 3 * _HID), lambda i: (0, 0)),
            pl.BlockSpec((1, _HID), lambda i: (0, 0)),
            pl.BlockSpec((3 * _HID, _HID), lambda i: (0, 0)),
            pl.BlockSpec((_HID, 1), lambda i: (0, 0)),
            pl.BlockSpec((1, 1), lambda i: (0, 0)),
        ],
        out_specs=[
            pl.BlockSpec((_EB, 1), lambda i: (i, 0)),
            pl.BlockSpec((1, 16), lambda i: (0, 0)),
        ],
        out_shape=[
            jax.ShapeDtypeStruct((_N_EDGES, 1), jnp.float32),
            jax.ShapeDtypeStruct((1, 16), jnp.float32),
        ],
    )(edge_tokens, tails, st_row, en_row, q, lng_col, lnb_row, b1_row, W1,
      W2, b2_2d)


_CHS = 80
_NCHS = _EPT // _CHS


def _sc_denom(lg2, tgt3, gmax16):
    @functools.partial(
        pl.kernel,
        mesh=_sc_mesh(),
        out_type=jax.ShapeDtypeStruct((2, _NPAD), jnp.float32),
        scratch_types=[
            pltpu.VMEM((_EPT,), jnp.float32),
            pltpu.VMEM((_NCHS, _CHS), jnp.int32),
            pltpu.VMEM((16,), jnp.float32),
            pltpu.VMEM((_NPAD,), jnp.float32),
            pltpu.VMEM_SHARED((_NPAD,), jnp.float32),
        ],
    )
    def k(lg_hbm, tgt_hbm, gm_hbm, out_hbm, lg_v, ix_v, gm_v, z_v, acc_sh):
        wid = _wid()
        core = lax.axis_index("c")
        sid = lax.axis_index("s")
        pltpu.sync_copy(lg_hbm.at[wid], lg_v)
        pltpu.sync_copy(tgt_hbm.at[wid], ix_v)
        pltpu.sync_copy(gm_hbm, gm_v)
        m = gm_v[...]

        @pl.when(sid == 0)
        def _():
            zero = jnp.zeros((16,), jnp.float32)

            def zb(i, c):
                z_v[pl.ds(i * 16, 16)] = zero
                return c

            lax.fori_loop(0, _NPAD // 16, zb, 0)
            pltpu.sync_copy(z_v, acc_sh)

        def eb(j, c):
            v = lg_v[pl.ds(j * 16, 16)]
            lg_v[pl.ds(j * 16, 16)] = jnp.exp(v - m)
            return c

        lax.fori_loop(0, _EPT // 16, eb, 0)
        plsc.subcore_barrier()

        def sb(ci, c):
            pltpu.sync_copy(
                lg_v.at[pl.ds(ci * _CHS, _CHS)],
                acc_sh.at[ix_v.at[ci]],
                add=True,
            )
            return c

        lax.fori_loop(0, _NCHS, sb, 0)
        plsc.subcore_barrier()

        @pl.when(sid == 0)
        def _():
            pltpu.sync_copy(acc_sh, out_hbm.at[core])

    return k(lg2, tgt3, gmax16)


def _tc_nodeoff(parts, gmax):
    def body(p_ref, m_ref, o_ref):
        den = jnp.sum(p_ref[...], axis=0, keepdims=True)
        o_ref[...] = jnp.max(m_ref[...]) + jnp.log(den)

    return pl.pallas_call(
        body, out_shape=jax.ShapeDtypeStruct((1, _NPAD), jnp.float32)
    )(parts, gmax)


def _sc_graphsum(nodeoff, tgt3, lg2, eb3, sel3):
    @functools.partial(
        pl.kernel,
        mesh=_sc_mesh(),
        out_type=jax.ShapeDtypeStruct((2, 512), jnp.float32),
        scratch_types=[
            pltpu.VMEM((_NCHS, _CHS), jnp.int32),
            pltpu.VMEM_SHARED((_NPAD,), jnp.float32),
            pltpu.VMEM((_EPT,), jnp.float32),
            pltpu.VMEM((_EPT,), jnp.float32),
            pltpu.VMEM((_NCHS, _CHS), jnp.float32),
            pltpu.VMEM((_NCHS, _CHS), jnp.float32),
            pltpu.VMEM((_NCHS, _CHS), jnp.int32),
            pltpu.VMEM((_NCHS, _CHS), jnp.int32),
            pltpu.VMEM((512,), jnp.float32),
            pltpu.VMEM_SHARED((512,), jnp.float32),
        ],
    )
    def k(off_hbm, tgt_hbm, lg_hbm, eb_hbm, sel_hbm, out_hbm,
          idx_v, off_sh, ofs_v, lg_v, lps_v, sel_v, kv_v, kv2_v, z_v, acc_sh):
        wid = _wid()
        core = lax.axis_index("c")
        sid = lax.axis_index("s")
        pltpu.sync_copy(tgt_hbm.at[wid], idx_v)
        pltpu.sync_copy(lg_hbm.at[wid], lg_v)
        pltpu.sync_copy(eb_hbm.at[wid], kv_v)
        pltpu.sync_copy(sel_hbm.at[wid], sel_v)

        @pl.when(sid == 0)
        def _():
            pltpu.sync_copy(off_hbm, off_sh)
            zero = jnp.zeros((16,), jnp.float32)

            def zb(i, c):
                z_v[pl.ds(i * 16, 16)] = zero
                return c

            lax.fori_loop(0, 512 // 16, zb, 0)
            pltpu.sync_copy(z_v, acc_sh)

        plsc.subcore_barrier()

        def gb(ci, c):
            pltpu.sync_copy(
                off_sh.at[idx_v.at[ci]], ofs_v.at[pl.ds(ci * _CHS, _CHS)]
            )
            return c

        lax.fori_loop(0, _NCHS, gb, 0)

        kbase = sid * 16

        def cb(j, c):
            ci = j // (_CHS // 16)
            l = (j % (_CHS // 16)) * 16
            v = lg_v[pl.ds(j * 16, 16)]
            o = ofs_v[pl.ds(j * 16, 16)]
            s = sel_v[ci, pl.ds(l, 16)]
            g = kv_v[ci, pl.ds(l, 16)]
            lps_v[ci, pl.ds(l, 16)] = (v - o) * s
            kv_v[ci, pl.ds(l, 16)] = g + kbase
            kv2_v[ci, pl.ds(l, 16)] = g + (kbase + 256)
            return c

        lax.fori_loop(0, _EPT // 16, cb, 0)

        def sb(ci, c):
            pltpu.sync_copy(lps_v.at[ci], acc_sh.at[kv_v.at[ci]], add=True)
            pltpu.sync_copy(sel_v.at[ci], acc_sh.at[kv2_v.at[ci]], add=True)
            return c

        lax.fori_loop(0, _NCHS, sb, 0)
        plsc.subcore_barrier()

        @pl.when(sid == 0)
        def _():
            pltpu.sync_copy(acc_sh, out_hbm.at[core])

    return k(nodeoff, tgt3, lg2, eb3, sel3)


def _tc_final(parts):
    def body(p_ref, o1_ref, o2_ref):
        f32 = jnp.float32
        x = p_ref[...]
        tot = (jnp.sum(x[0:16], axis=0, keepdims=True)
               + jnp.sum(x[32:48], axis=0, keepdims=True))
        cnt = (jnp.sum(x[16:32], axis=0, keepdims=True)
               + jnp.sum(x[48:64], axis=0, keepdims=True))
        has = cnt > 0.0
        nll = jnp.where(has, -tot, 0.0)
        ngr = jnp.maximum(jnp.sum(has.astype(f32)), 1.0)
        o1_ref[...] = tot
        o2_ref[...] = (jnp.sum(nll) / ngr).reshape(1, 1)

    return pl.pallas_call(
        body,
        out_shape=[
            jax.ShapeDtypeStruct((1, _NG), jnp.float32),
            jax.ShapeDtypeStruct((1, 1), jnp.float32),
        ],
    )(parts)


def kernel(edge_tokens, node_tokens, question_tokens, ln_g, ln_b, W1, b1, W2,
           b2, edge_batch, selected_mask, edge_index):
    tgt = edge_index[1]
    tails = _sc_gather_tails(node_tokens, tgt.reshape(_NW, _NCH, _CH))
    starts = jnp.searchsorted(
        edge_batch, jnp.arange(_NG, dtype=edge_batch.dtype)
    ).astype(jnp.int32)
    ends = jnp.concatenate(
        [starts[1:], jnp.full((1,), _N_EDGES, jnp.int32)]
    )
    logits, gmax = _tc_logits(
        edge_tokens, tails,
        starts.reshape(1, _NG),
        ends.reshape(1, _NG),
        question_tokens,
        ln_g.reshape(3 * _HID, 1),
        ln_b.reshape(1, 3 * _HID),
        W1,
        b1.reshape(1, _HID),
        W2,
        b2.reshape(1, 1),
    )
    lg2 = logits.reshape(_NW, _EPT)
    tgt3 = tgt.reshape(_NW, _NCHS, _CHS)
    denp = _sc_denom(lg2, tgt3, gmax.reshape(16))
    nodeoff = _tc_nodeoff(denp, gmax)
    parts = _sc_graphsum(
        nodeoff.reshape(_NPAD), tgt3, lg2,
        edge_batch.reshape(_NW, _NCHS, _CHS),
        selected_mask.astype(jnp.float32).reshape(_NW, _NCHS, _CHS),
    )
    logpb, nll = _tc_final(parts.reshape(64, _NG))
    return logpb.reshape(_NG), nll.reshape(())


# speedup vs baseline: 12.5431x; 1.0195x over previous
import functools

import jax
import jax.numpy as jnp
from jax import lax
from jax.experimental import pallas as pl
from jax.experimental.pallas import tpu as pltpu
from jax.experimental.pallas import tpu_sc as plsc

_N_NODES = 10000
_N_EDGES = 320000
_HID = 128
_NG = 16

_NW = 32
_EPT = _N_EDGES // _NW
_CH = 40
_NCH = _EPT // _CH
_NPAD = 10240
_EB = 2560
_NB = _N_EDGES // _EB


def _sc_mesh():
    return plsc.VectorSubcoreMesh(
        core_axis_name="c", subcore_axis_name="s", num_cores=2, num_subcores=16
    )


def _wid():
    return lax.axis_index("s") * 2 + lax.axis_index("c")


def _sc_gather_tails(node_tokens, tgt3):
    @functools.partial(
        pl.kernel,
        mesh=_sc_mesh(),
        out_type=jax.ShapeDtypeStruct((_N_EDGES, _HID), jnp.float32),
        scratch_types=[
            pltpu.VMEM((_NCH, _CH), jnp.int32),
            pltpu.VMEM((_CH, _HID), jnp.float32),
            pltpu.VMEM((_CH, _HID), jnp.float32),
            pltpu.SemaphoreType.DMA,
            pltpu.SemaphoreType.DMA,
        ],
    )
    def k(node_hbm, tgt_hbm, out_hbm, idx_v, buf0, buf1, sem0, sem1):
        wid = _wid()
        base = wid * _EPT
        pltpu.sync_copy(tgt_hbm.at[wid], idx_v)
        pltpu.async_copy(node_hbm.at[idx_v.at[0]], buf0, sem0)
        pltpu.async_copy(node_hbm.at[idx_v.at[1]], buf1, sem1)

        def step(c, buf, sem):
            pltpu.make_async_copy(node_hbm.at[idx_v.at[c]], buf, sem).wait()
            pltpu.sync_copy(buf, out_hbm.at[pl.ds(base + c * _CH, _CH)])

            @pl.when(c + 2 < _NCH)
            def _():
                pltpu.async_copy(node_hbm.at[idx_v.at[c + 2]], buf, sem)

        def body(c, carry):
            @pl.when(c % 2 == 0)
            def _():
                step(c, buf0, sem0)

            @pl.when(c % 2 == 1)
            def _():
                step(c, buf1, sem1)

            return carry

        lax.fori_loop(0, _NCH, body, 0)

    return k(node_tokens, tgt3)


def _tc_logits(edge_tokens, tails, st_row, en_row, q, lng_col, lnb_row, W1,
               b1_row, W2, b2_2d):
    def body(et_ref, tl_ref, st_ref, en_ref, q_ref, lng_ref, lnb_ref, b1_ref,
             w1_ref, w2_ref, b2_ref, out_ref):
        f32 = jnp.float32
        e = et_ref[...]
        t = tl_ref[...]
        w1 = w1_ref[...]
        w1g = w1 * lng_ref[...]
        wbar = jnp.sum(w1g, axis=0, keepdims=True)
        hb = jnp.dot(lnb_ref[...], w1, preferred_element_type=f32) + b1_ref[...]
        w1e = w1g[:_HID]
        w1q = w1g[_HID:2 * _HID]
        w1t = w1g[2 * _HID:]
        qt = q_ref[...]
        qp = jnp.dot(qt, w1q, preferred_element_type=f32)
        sq = jnp.sum(qt, axis=1, keepdims=True)
        sqq = jnp.sum(qt * qt, axis=1, keepdims=True)
        eid = pl.program_id(0) * _EB + lax.broadcasted_iota(
            jnp.int32, (_EB, 1), 0
        )
        onehot = ((eid >= st_ref[...]) & (eid < en_ref[...])).astype(f32)
        qpe = jnp.dot(onehot, qp, preferred_element_type=f32)
        sqe = jnp.dot(onehot, sq, preferred_element_type=f32)
        sqqe = jnp.dot(onehot, sqq, preferred_element_type=f32)
        xw = (jnp.dot(e, w1e, preferred_element_type=f32)
              + jnp.dot(t, w1t, preferred_element_type=f32) + qpe)
        ones_col = jnp.ones((_HID, 1), f32)
        u = e + t
        v = e * e + t * t
        suv = jnp.dot(u, ones_col, preferred_element_type=f32)
        svv = jnp.dot(v, ones_col, preferred_element_type=f32)
        mu = (suv + sqe) * (1.0 / 384.0)
        var = (svv + sqqe) * (1.0 / 384.0) - mu * mu
        r = lax.rsqrt(var + 1e-5)
        h = xw * r - (mu * r) * wbar + hb
        gel = 0.5 * h * (1.0 + lax.erf(h * 0.7071067811865476))
        lg = jnp.dot(gel, w2_ref[...], preferred_element_type=f32) \
            + b2_ref[...]
        out_ref[...] = lg

    return pl.pallas_call(
        body,
        grid=(_NB,),
        in_specs=[
            pl.BlockSpec((_EB, _HID), lambda i: (i, 0)),
            pl.BlockSpec((_EB, _HID), lambda i: (i, 0)),
            pl.BlockSpec((1, _NG), lambda i: (0, 0)),
            pl.BlockSpec((1, _NG), lambda i: (0, 0)),
            pl.BlockSpec((_NG, _HID), lambda i: (0, 0)),
            pl.BlockSpec((3 * _HID, 1), lambda i: (0, 0)),
            pl.BlockSpec((1, 3 * _HID), lambda i: (0, 0)),
            pl.BlockSpec((1, _HID), lambda i: (0, 0)),
            pl.BlockSpec((3 * _HID, _HID), lambda i: (0, 0)),
            pl.BlockSpec((_HID, 1), lambda i: (0, 0)),
            pl.BlockSpec((1, 1), lambda i: (0, 0)),
        ],
        out_specs=pl.BlockSpec((_EB, 1), lambda i: (i, 0)),
        out_shape=jax.ShapeDtypeStruct((_N_EDGES, 1), jnp.float32),
    )(edge_tokens, tails, st_row, en_row, q, lng_col, lnb_row, b1_row, W1,
      W2, b2_2d)


_CHS = 80
_NCHS = _EPT // _CHS


def _sc_denom(lg2, tgt3):
    @functools.partial(
        pl.kernel,
        mesh=_sc_mesh(),
        out_type=(
            jax.ShapeDtypeStruct((2, _NPAD), jnp.float32),
            jax.ShapeDtypeStruct((2, 16), jnp.float32),
        ),
        scratch_types=[
            pltpu.VMEM((_EPT,), jnp.float32),
            pltpu.VMEM((_NCHS, _CHS), jnp.int32),
            pltpu.VMEM((16,), jnp.float32),
            pltpu.VMEM((_NPAD,), jnp.float32),
            pltpu.VMEM_SHARED((_NPAD,), jnp.float32),
            pltpu.VMEM_SHARED((256,), jnp.float32),
        ],
    )
    def k(lg_hbm, tgt_hbm, out_hbm, gm_hbm, lg_v, ix_v, gm_v, z_v, acc_sh,
          max_sh):
        wid = _wid()
        core = lax.axis_index("c")
        sid = lax.axis_index("s")
        pltpu.sync_copy(lg_hbm.at[wid], lg_v)
        pltpu.sync_copy(tgt_hbm.at[wid], ix_v)

        gm_v[...] = jnp.full((16,), -1e30, jnp.float32)

        def mb(j, c):
            gm_v[...] = jnp.maximum(gm_v[...], lg_v[pl.ds(j * 16, 16)])
            return c

        lax.fori_loop(0, _EPT // 16, mb, 0)
        pltpu.sync_copy(gm_v, max_sh.at[pl.ds(sid * 16, 16)])

        @pl.when(sid == 0)
        def _():
            zero = jnp.zeros((16,), jnp.float32)

            def zb(i, c):
                z_v[pl.ds(i * 16, 16)] = zero
                return c

            lax.fori_loop(0, _NPAD // 16, zb, 0)
            pltpu.sync_copy(z_v, acc_sh)

        plsc.subcore_barrier()
        pltpu.sync_copy(max_sh, z_v.at[pl.ds(0, 256)])
        mm = z_v[pl.ds(0, 16)]
        for r in range(1, 16):
            mm = jnp.maximum(mm, z_v[pl.ds(r * 16, 16)])
        s = mm[0]
        for i in range(1, 16):
            s = jnp.maximum(s, mm[i])
        m = jnp.full((16,), s, jnp.float32)
        gm_v[...] = m

        def eb(j, c):
            v = lg_v[pl.ds(j * 16, 16)]
            lg_v[pl.ds(j * 16, 16)] = jnp.exp(v - m)
            return c

        lax.fori_loop(0, _EPT // 16, eb, 0)

        def sb(ci, c):
            pltpu.sync_copy(
                lg_v.at[pl.ds(ci * _CHS, _CHS)],
                acc_sh.at[ix_v.at[ci]],
                add=True,
            )
            return c

        lax.fori_loop(0, _NCHS, sb, 0)
        plsc.subcore_barrier()

        @pl.when(sid == 0)
        def _():
            pltpu.sync_copy(acc_sh, out_hbm.at[core])
            pltpu.sync_copy(gm_v, gm_hbm.at[core])

    return k(lg2, tgt3)


def _tc_nodeoff(parts, gmax2):
    def body(p_ref, m_ref, o_ref):
        m0 = jnp.max(m_ref[0:1, :])
        m1 = jnp.max(m_ref[1:2, :])
        M = jnp.maximum(m0, m1)
        den = (p_ref[0:1, :] * jnp.exp(m0 - M)
               + p_ref[1:2, :] * jnp.exp(m1 - M))
        o_ref[...] = M + jnp.log(den)

    return pl.pallas_call(
        body, out_shape=jax.ShapeDtypeStruct((1, _NPAD), jnp.float32)
    )(parts, gmax2)


def _sc_graphsum(nodeoff, tgt3, lg2, eb3, sel3):
    @functools.partial(
        pl.kernel,
        mesh=_sc_mesh(),
        out_type=jax.ShapeDtypeStruct((2, 512), jnp.float32),
        scratch_types=[
            pltpu.VMEM((_NCHS, _CHS), jnp.int32),
            pltpu.VMEM_SHARED((_NPAD,), jnp.float32),
            pltpu.VMEM((_EPT,), jnp.float32),
            pltpu.VMEM((_EPT,), jnp.float32),
            pltpu.VMEM((_NCHS, _CHS), jnp.float32),
            pltpu.VMEM((_NCHS, _CHS), jnp.float32),
            pltpu.VMEM((_NCHS, _CHS), jnp.int32),
            pltpu.VMEM((_NCHS, _CHS), jnp.int32),
            pltpu.VMEM((512,), jnp.float32),
            pltpu.VMEM_SHARED((512,), jnp.float32),
        ],
    )
    def k(off_hbm, tgt_hbm, lg_hbm, eb_hbm, sel_hbm, out_hbm,
          idx_v, off_sh, ofs_v, lg_v, lps_v, sel_v, kv_v, kv2_v, z_v, acc_sh):
        wid = _wid()
        core = lax.axis_index("c")
        sid = lax.axis_index("s")
        pltpu.sync_copy(tgt_hbm.at[wid], idx_v)
        pltpu.sync_copy(lg_hbm.at[wid], lg_v)
        pltpu.sync_copy(eb_hbm.at[wid], kv_v)
        pltpu.sync_copy(sel_hbm.at[wid], sel_v)

        @pl.when(sid == 0)
        def _():
            pltpu.sync_copy(off_hbm, off_sh)
            zero = jnp.zeros((16,), jnp.float32)

            def zb(i, c):
                z_v[pl.ds(i * 16, 16)] = zero
                return c

            lax.fori_loop(0, 512 // 16, zb, 0)
            pltpu.sync_copy(z_v, acc_sh)

        plsc.subcore_barrier()

        def gb(ci, c):
            pltpu.sync_copy(
                off_sh.at[idx_v.at[ci]], ofs_v.at[pl.ds(ci * _CHS, _CHS)]
            )
            return c

        lax.fori_loop(0, _NCHS, gb, 0)

        kbase = sid * 16

        def cb(j, c):
            ci = j // (_CHS // 16)
            l = (j % (_CHS // 16)) * 16
            v = lg_v[pl.ds(j * 16, 16)]
            o = ofs_v[pl.ds(j * 16, 16)]
            s = sel_v[ci, pl.ds(l, 16)]
            g = kv_v[ci, pl.ds(l, 16)]
            lps_v[ci, pl.ds(l, 16)] = (v - o) * s
            kv_v[ci, pl.ds(l, 16)] = g + kbase
            kv2_v[ci, pl.ds(l, 16)] = g + (kbase + 256)
            return c

        lax.fori_loop(0, _EPT // 16, cb, 0)

        def sb(ci, c):
            pltpu.sync_copy(lps_v.at[ci], acc_sh.at[kv_v.at[ci]], add=True)
            pltpu.sync_copy(sel_v.at[ci], acc_sh.at[kv2_v.at[ci]], add=True)
            return c

        lax.fori_loop(0, _NCHS, sb, 0)
        plsc.subcore_barrier()

        @pl.when(sid == 0)
        def _():
            pltpu.sync_copy(acc_sh, out_hbm.at[core])

    return k(nodeoff, tgt3, lg2, eb3, sel3)


def _tc_final(parts):
    def body(p_ref, o1_ref, o2_ref):
        f32 = jnp.float32
        x = p_ref[...]
        tot = (jnp.sum(x[0:16], axis=0, keepdims=True)
               + jnp.sum(x[32:48], axis=0, keepdims=True))
        cnt = (jnp.sum(x[16:32], axis=0, keepdims=True)
               + jnp.sum(x[48:64], axis=0, keepdims=True))
        has = cnt > 0.0
        nll = jnp.where(has, -tot, 0.0)
        ngr = jnp.maximum(jnp.sum(has.astype(f32)), 1.0)
        o1_ref[...] = tot
        o2_ref[...] = (jnp.sum(nll) / ngr).reshape(1, 1)

    return pl.pallas_call(
        body,
        out_shape=[
            jax.ShapeDtypeStruct((1, _NG), jnp.float32),
            jax.ShapeDtypeStruct((1, 1), jnp.float32),
        ],
    )(parts)


def kernel(edge_tokens, node_tokens, question_tokens, ln_g, ln_b, W1, b1, W2,
           b2, edge_batch, selected_mask, edge_index):
    tgt = edge_index[1]
    tails = _sc_gather_tails(node_tokens, tgt.reshape(_NW, _NCH, _CH))
    starts = jnp.searchsorted(
        edge_batch, jnp.arange(_NG, dtype=edge_batch.dtype)
    ).astype(jnp.int32)
    ends = jnp.concatenate(
        [starts[1:], jnp.full((1,), _N_EDGES, jnp.int32)]
    )
    logits = _tc_logits(
        edge_tokens, tails,
        starts.reshape(1, _NG),
        ends.reshape(1, _NG),
        question_tokens,
        ln_g.reshape(3 * _HID, 1),
        ln_b.reshape(1, 3 * _HID),
        W1,
        b1.reshape(1, _HID),
        W2,
        b2.reshape(1, 1),
    )
    lg2 = logits.reshape(_NW, _EPT)
    tgt3 = tgt.reshape(_NW, _NCHS, _CHS)
    denp, gmax2 = _sc_denom(lg2, tgt3)
    nodeoff = _tc_nodeoff(denp, gmax2)
    parts = _sc_graphsum(
        nodeoff.reshape(_NPAD), tgt3, lg2,
        edge_batch.reshape(_NW, _NCHS, _CHS),
        selected_mask.astype(jnp.float32).reshape(_NW, _NCHS, _CHS),
    )
    logpb, nll = _tc_final(parts.reshape(64, _NG))
    return logpb.reshape(_NG), nll.reshape(())


# speedup vs baseline: 13.4941x; 1.0758x over previous
import functools

import jax
import jax.numpy as jnp
from jax import lax
from jax.experimental import pallas as pl
from jax.experimental.pallas import tpu as pltpu
from jax.experimental.pallas import tpu_sc as plsc

_N_NODES = 10000
_N_EDGES = 320000
_HID = 128
_NG = 16

_NW = 32
_EPT = _N_EDGES // _NW
_CH = 40
_NCH = _EPT // _CH
_NPAD = 10240
_EB = 2560
_NB = _N_EDGES // _EB


def _sc_mesh():
    return plsc.VectorSubcoreMesh(
        core_axis_name="c", subcore_axis_name="s", num_cores=2, num_subcores=16
    )


def _wid():
    return lax.axis_index("s") * 2 + lax.axis_index("c")


def _sc_gather_tails(node_tokens, tgt3):
    @functools.partial(
        pl.kernel,
        mesh=_sc_mesh(),
        out_type=jax.ShapeDtypeStruct((_N_EDGES, _HID), jnp.float32),
        scratch_types=[
            pltpu.VMEM((_NCH, _CH), jnp.int32),
            pltpu.VMEM((_CH, _HID), jnp.float32),
            pltpu.VMEM((_CH, _HID), jnp.float32),
            pltpu.VMEM((_CH, _HID), jnp.float32),
            pltpu.VMEM((_CH, _HID), jnp.float32),
            pltpu.SemaphoreType.DMA,
            pltpu.SemaphoreType.DMA,
            pltpu.SemaphoreType.DMA,
            pltpu.SemaphoreType.DMA,
            pltpu.SemaphoreType.DMA,
            pltpu.SemaphoreType.DMA,
            pltpu.SemaphoreType.DMA,
            pltpu.SemaphoreType.DMA,
        ],
    )
    def k(node_hbm, tgt_hbm, out_hbm, idx_v, b0, b1, b2, b3,
          g0, g1, g2, g3, w0, w1, w2, w3):
        wid = _wid()
        base = wid * _EPT
        bufs = (b0, b1, b2, b3)
        gsems = (g0, g1, g2, g3)
        wsems = (w0, w1, w2, w3)
        pltpu.sync_copy(tgt_hbm.at[wid], idx_v)
        pltpu.async_copy(node_hbm.at[idx_v.at[0]], b0, g0)
        pltpu.async_copy(node_hbm.at[idx_v.at[1]], b1, g1)

        def step(c, b):
            bp = (b + 2) % 4

            @pl.when(c + 2 < _NCH)
            def _():
                @pl.when(c >= 2)
                def _():
                    pltpu.make_async_copy(
                        bufs[bp],
                        out_hbm.at[pl.ds(base + (c - 2) * _CH, _CH)],
                        wsems[bp],
                    ).wait()

                pltpu.async_copy(node_hbm.at[idx_v.at[c + 2]], bufs[bp],
                                 gsems[bp])

            pltpu.make_async_copy(node_hbm.at[idx_v.at[c]], bufs[b],
                                  gsems[b]).wait()
            pltpu.async_copy(bufs[b], out_hbm.at[pl.ds(base + c * _CH, _CH)],
                             wsems[b])

        def body(c, carry):
            for b in range(4):
                @pl.when(c % 4 == b)
                def _(b=b):
                    step(c, b)

            return carry

        lax.fori_loop(0, _NCH, body, 0)
        for b in range(4):
            cb = _NCH - 4 + ((b - _NCH) % 4)
            pltpu.make_async_copy(
                bufs[b], out_hbm.at[pl.ds(base + cb * _CH, _CH)], wsems[b]
            ).wait()

    return k(node_tokens, tgt3)


def _tc_logits(edge_tokens, tails, st_row, en_row, q, lng_col, lnb_row, W1,
               b1_row, W2, b2_2d):
    def body(et_ref, tl_ref, st_ref, en_ref, q_ref, lng_ref, lnb_ref, b1_ref,
             w1_ref, w2_ref, b2_ref, out_ref):
        f32 = jnp.float32
        e = et_ref[...]
        t = tl_ref[...]
        w1 = w1_ref[...]
        w1g = w1 * lng_ref[...]
        wbar = jnp.sum(w1g, axis=0, keepdims=True)
        hb = jnp.dot(lnb_ref[...], w1, preferred_element_type=f32) + b1_ref[...]
        w1e = w1g[:_HID]
        w1q = w1g[_HID:2 * _HID]
        w1t = w1g[2 * _HID:]
        qt = q_ref[...]
        qp = jnp.dot(qt, w1q, preferred_element_type=f32)
        sq = jnp.sum(qt, axis=1, keepdims=True)
        sqq = jnp.sum(qt * qt, axis=1, keepdims=True)
        eid = pl.program_id(0) * _EB + lax.broadcasted_iota(
            jnp.int32, (_EB, 1), 0
        )
        onehot = ((eid >= st_ref[...]) & (eid < en_ref[...])).astype(f32)
        qpe = jnp.dot(onehot, qp, preferred_element_type=f32)
        sqe = jnp.dot(onehot, sq, preferred_element_type=f32)
        sqqe = jnp.dot(onehot, sqq, preferred_element_type=f32)
        xw = (jnp.dot(e, w1e, preferred_element_type=f32)
              + jnp.dot(t, w1t, preferred_element_type=f32) + qpe)
        ones_col = jnp.ones((_HID, 1), f32)
        u = e + t
        v = e * e + t * t
        suv = jnp.dot(u, ones_col, preferred_element_type=f32)
        svv = jnp.dot(v, ones_col, preferred_element_type=f32)
        mu = (suv + sqe) * (1.0 / 384.0)
        var = (svv + sqqe) * (1.0 / 384.0) - mu * mu
        r = lax.rsqrt(var + 1e-5)
        h = xw * r - (mu * r) * wbar + hb
        gel = 0.5 * h * (1.0 + lax.erf(h * 0.7071067811865476))
        lg = jnp.dot(gel, w2_ref[...], preferred_element_type=f32) \
            + b2_ref[...]
        out_ref[...] = lg

    return pl.pallas_call(
        body,
        grid=(_NB,),
        in_specs=[
            pl.BlockSpec((_EB, _HID), lambda i: (i, 0)),
            pl.BlockSpec((_EB, _HID), lambda i: (i, 0)),
            pl.BlockSpec((1, _NG), lambda i: (0, 0)),
            pl.BlockSpec((1, _NG), lambda i: (0, 0)),
            pl.BlockSpec((_NG, _HID), lambda i: (0, 0)),
            pl.BlockSpec((3 * _HID, 1), lambda i: (0, 0)),
            pl.BlockSpec((1, 3 * _HID), lambda i: (0, 0)),
            pl.BlockSpec((1, _HID), lambda i: (0, 0)),
            pl.BlockSpec((3 * _HID, _HID), lambda i: (0, 0)),
            pl.BlockSpec((_HID, 1), lambda i: (0, 0)),
            pl.BlockSpec((1, 1), lambda i: (0, 0)),
        ],
        out_specs=pl.BlockSpec((_EB, 1), lambda i: (i, 0)),
        out_shape=jax.ShapeDtypeStruct((_N_EDGES, 1), jnp.float32),
    )(edge_tokens, tails, st_row, en_row, q, lng_col, lnb_row, b1_row, W1,
      W2, b2_2d)


_CHS = 80
_NCHS = _EPT // _CHS


def _sc_denom(lg2, tgt3):
    @functools.partial(
        pl.kernel,
        mesh=_sc_mesh(),
        out_type=(
            jax.ShapeDtypeStruct((2, _NPAD), jnp.float32),
            jax.ShapeDtypeStruct((2, 16), jnp.float32),
        ),
        scratch_types=[
            pltpu.VMEM((_EPT,), jnp.float32),
            pltpu.VMEM((_NCHS, _CHS), jnp.int32),
            pltpu.VMEM((16,), jnp.float32),
            pltpu.VMEM((_NPAD,), jnp.float32),
            pltpu.VMEM_SHARED((_NPAD,), jnp.float32),
            pltpu.VMEM_SHARED((256,), jnp.float32),
        ],
    )
    def k(lg_hbm, tgt_hbm, out_hbm, gm_hbm, lg_v, ix_v, gm_v, z_v, acc_sh,
          max_sh):
        wid = _wid()
        core = lax.axis_index("c")
        sid = lax.axis_index("s")
        pltpu.sync_copy(lg_hbm.at[wid], lg_v)
        pltpu.sync_copy(tgt_hbm.at[wid], ix_v)

        gm_v[...] = jnp.full((16,), -1e30, jnp.float32)

        def mb(j, c):
            gm_v[...] = jnp.maximum(gm_v[...], lg_v[pl.ds(j * 16, 16)])
            return c

        lax.fori_loop(0, _EPT // 16, mb, 0)
        pltpu.sync_copy(gm_v, max_sh.at[pl.ds(sid * 16, 16)])

        @pl.when(sid == 0)
        def _():
            zero = jnp.zeros((16,), jnp.float32)

            def zb(i, c):
                z_v[pl.ds(i * 16, 16)] = zero
                return c

            lax.fori_loop(0, _NPAD // 16, zb, 0)
            pltpu.sync_copy(z_v, acc_sh)

        plsc.subcore_barrier()
        pltpu.sync_copy(max_sh, z_v.at[pl.ds(0, 256)])
        mm = z_v[pl.ds(0, 16)]
        for r in range(1, 16):
            mm = jnp.maximum(mm, z_v[pl.ds(r * 16, 16)])
        s = mm[0]
        for i in range(1, 16):
            s = jnp.maximum(s, mm[i])
        m = jnp.full((16,), s, jnp.float32)
        gm_v[...] = m

        def eb(j, c):
            v = lg_v[pl.ds(j * 16, 16)]
            lg_v[pl.ds(j * 16, 16)] = jnp.exp(v - m)
            return c

        lax.fori_loop(0, _EPT // 16, eb, 0)

        def sb(ci, c):
            pltpu.sync_copy(
                lg_v.at[pl.ds(ci * _CHS, _CHS)],
                acc_sh.at[ix_v.at[ci]],
                add=True,
            )
            return c

        lax.fori_loop(0, _NCHS, sb, 0)
        plsc.subcore_barrier()

        @pl.when(sid == 0)
        def _():
            pltpu.sync_copy(acc_sh, out_hbm.at[core])
            pltpu.sync_copy(gm_v, gm_hbm.at[core])

    return k(lg2, tgt3)


def _tc_nodeoff(parts, gmax2):
    def body(p_ref, m_ref, o_ref):
        m0 = jnp.max(m_ref[0:1, :])
        m1 = jnp.max(m_ref[1:2, :])
        M = jnp.maximum(m0, m1)
        den = (p_ref[0:1, :] * jnp.exp(m0 - M)
               + p_ref[1:2, :] * jnp.exp(m1 - M))
        o_ref[...] = M + jnp.log(den)

    return pl.pallas_call(
        body, out_shape=jax.ShapeDtypeStruct((1, _NPAD), jnp.float32)
    )(parts, gmax2)


def _sc_graphsum(nodeoff, tgt3, lg2, eb3, sel3):
    @functools.partial(
        pl.kernel,
        mesh=_sc_mesh(),
        out_type=jax.ShapeDtypeStruct((2, 512), jnp.float32),
        scratch_types=[
            pltpu.VMEM((_NCHS, _CHS), jnp.int32),
            pltpu.VMEM_SHARED((_NPAD,), jnp.float32),
            pltpu.VMEM((_EPT,), jnp.float32),
            pltpu.VMEM((_EPT,), jnp.float32),
            pltpu.VMEM((_NCHS, _CHS), jnp.float32),
            pltpu.VMEM((_NCHS, _CHS), jnp.float32),
            pltpu.VMEM((_NCHS, _CHS), jnp.int32),
            pltpu.VMEM((_NCHS, _CHS), jnp.int32),
            pltpu.VMEM((512,), jnp.float32),
            pltpu.VMEM_SHARED((512,), jnp.float32),
        ],
    )
    def k(off_hbm, tgt_hbm, lg_hbm, eb_hbm, sel_hbm, out_hbm,
          idx_v, off_sh, ofs_v, lg_v, lps_v, sel_v, kv_v, kv2_v, z_v, acc_sh):
        wid = _wid()
        core = lax.axis_index("c")
        sid = lax.axis_index("s")
        pltpu.sync_copy(tgt_hbm.at[wid], idx_v)
        pltpu.sync_copy(lg_hbm.at[wid], lg_v)
        pltpu.sync_copy(eb_hbm.at[wid], kv_v)
        pltpu.sync_copy(sel_hbm.at[wid], sel_v)

        @pl.when(sid == 0)
        def _():
            pltpu.sync_copy(off_hbm, off_sh)
            zero = jnp.zeros((16,), jnp.float32)

            def zb(i, c):
                z_v[pl.ds(i * 16, 16)] = zero
                return c

            lax.fori_loop(0, 512 // 16, zb, 0)
            pltpu.sync_copy(z_v, acc_sh)

        plsc.subcore_barrier()

        def gb(ci, c):
            pltpu.sync_copy(
                off_sh.at[idx_v.at[ci]], ofs_v.at[pl.ds(ci * _CHS, _CHS)]
            )
            return c

        lax.fori_loop(0, _NCHS, gb, 0)

        kbase = sid * 16

        def cb(j, c):
            ci = j // (_CHS // 16)
            l = (j % (_CHS // 16)) * 16
            v = lg_v[pl.ds(j * 16, 16)]
            o = ofs_v[pl.ds(j * 16, 16)]
            s = sel_v[ci, pl.ds(l, 16)]
            g = kv_v[ci, pl.ds(l, 16)]
            lps_v[ci, pl.ds(l, 16)] = (v - o) * s
            kv_v[ci, pl.ds(l, 16)] = g + kbase
            kv2_v[ci, pl.ds(l, 16)] = g + (kbase + 256)
            return c

        lax.fori_loop(0, _EPT // 16, cb, 0)

        def sb(ci, c):
            pltpu.sync_copy(lps_v.at[ci], acc_sh.at[kv_v.at[ci]], add=True)
            pltpu.sync_copy(sel_v.at[ci], acc_sh.at[kv2_v.at[ci]], add=True)
            return c

        lax.fori_loop(0, _NCHS, sb, 0)
        plsc.subcore_barrier()

        @pl.when(sid == 0)
        def _():
            pltpu.sync_copy(acc_sh, out_hbm.at[core])

    return k(nodeoff, tgt3, lg2, eb3, sel3)


def _tc_final(parts):
    def body(p_ref, o1_ref, o2_ref):
        f32 = jnp.float32
        x = p_ref[...]
        tot = (jnp.sum(x[0:16], axis=0, keepdims=True)
               + jnp.sum(x[32:48], axis=0, keepdims=True))
        cnt = (jnp.sum(x[16:32], axis=0, keepdims=True)
               + jnp.sum(x[48:64], axis=0, keepdims=True))
        has = cnt > 0.0
        nll = jnp.where(has, -tot, 0.0)
        ngr = jnp.maximum(jnp.sum(has.astype(f32)), 1.0)
        o1_ref[...] = tot
        o2_ref[...] = (jnp.sum(nll) / ngr).reshape(1, 1)

    return pl.pallas_call(
        body,
        out_shape=[
            jax.ShapeDtypeStruct((1, _NG), jnp.float32),
            jax.ShapeDtypeStruct((1, 1), jnp.float32),
        ],
    )(parts)


def kernel(edge_tokens, node_tokens, question_tokens, ln_g, ln_b, W1, b1, W2,
           b2, edge_batch, selected_mask, edge_index):
    tgt = edge_index[1]
    tails = _sc_gather_tails(node_tokens, tgt.reshape(_NW, _NCH, _CH))
    starts = jnp.searchsorted(
        edge_batch, jnp.arange(_NG, dtype=edge_batch.dtype)
    ).astype(jnp.int32)
    ends = jnp.concatenate(
        [starts[1:], jnp.full((1,), _N_EDGES, jnp.int32)]
    )
    logits = _tc_logits(
        edge_tokens, tails,
        starts.reshape(1, _NG),
        ends.reshape(1, _NG),
        question_tokens,
        ln_g.reshape(3 * _HID, 1),
        ln_b.reshape(1, 3 * _HID),
        W1,
        b1.reshape(1, _HID),
        W2,
        b2.reshape(1, 1),
    )
    lg2 = logits.reshape(_NW, _EPT)
    tgt3 = tgt.reshape(_NW, _NCHS, _CHS)
    denp, gmax2 = _sc_denom(lg2, tgt3)
    nodeoff = _tc_nodeoff(denp, gmax2)
    parts = _sc_graphsum(
        nodeoff.reshape(_NPAD), tgt3, lg2,
        edge_batch.reshape(_NW, _NCHS, _CHS),
        selected_mask.astype(jnp.float32).reshape(_NW, _NCHS, _CHS),
    )
    logpb, nll = _tc_final(parts.reshape(64, _NG))
    return logpb.reshape(_NG), nll.reshape(())


# speedup vs baseline: 13.7153x; 1.0164x over previous
import functools

import jax
import jax.numpy as jnp
from jax import lax
from jax.experimental import pallas as pl
from jax.experimental.pallas import tpu as pltpu
from jax.experimental.pallas import tpu_sc as plsc

_N_NODES = 10000
_N_EDGES = 320000
_HID = 128
_NG = 16

_NW = 32
_EPT = _N_EDGES // _NW
_CH = 80
_NCH = _EPT // _CH
_NPAD = 10240
_EB = 2560
_NB = _N_EDGES // _EB


def _sc_mesh():
    return plsc.VectorSubcoreMesh(
        core_axis_name="c", subcore_axis_name="s", num_cores=2, num_subcores=16
    )


def _wid():
    return lax.axis_index("s") * 2 + lax.axis_index("c")


def _sc_gather_tails(node_tokens, tgt3):
    @functools.partial(
        pl.kernel,
        mesh=_sc_mesh(),
        out_type=jax.ShapeDtypeStruct((_N_EDGES, _HID), jnp.float32),
        scratch_types=[
            pltpu.VMEM((_NCH, _CH), jnp.int32),
            pltpu.VMEM((_CH, _HID), jnp.float32),
            pltpu.VMEM((_CH, _HID), jnp.float32),
            pltpu.VMEM((_CH, _HID), jnp.float32),
            pltpu.VMEM((_CH, _HID), jnp.float32),
            pltpu.SemaphoreType.DMA,
            pltpu.SemaphoreType.DMA,
            pltpu.SemaphoreType.DMA,
            pltpu.SemaphoreType.DMA,
            pltpu.SemaphoreType.DMA,
            pltpu.SemaphoreType.DMA,
            pltpu.SemaphoreType.DMA,
            pltpu.SemaphoreType.DMA,
        ],
    )
    def k(node_hbm, tgt_hbm, out_hbm, idx_v, b0, b1, b2, b3,
          g0, g1, g2, g3, w0, w1, w2, w3):
        wid = _wid()
        base = wid * _EPT
        bufs = (b0, b1, b2, b3)
        gsems = (g0, g1, g2, g3)
        wsems = (w0, w1, w2, w3)
        pltpu.sync_copy(tgt_hbm.at[wid], idx_v)
        pltpu.async_copy(node_hbm.at[idx_v.at[0]], b0, g0)
        pltpu.async_copy(node_hbm.at[idx_v.at[1]], b1, g1)

        def step(c, b):
            bp = (b + 2) % 4

            @pl.when(c + 2 < _NCH)
            def _():
                @pl.when(c >= 2)
                def _():
                    pltpu.make_async_copy(
                        bufs[bp],
                        out_hbm.at[pl.ds(base + (c - 2) * _CH, _CH)],
                        wsems[bp],
                    ).wait()

                pltpu.async_copy(node_hbm.at[idx_v.at[c + 2]], bufs[bp],
                                 gsems[bp])

            pltpu.make_async_copy(node_hbm.at[idx_v.at[c]], bufs[b],
                                  gsems[b]).wait()
            pltpu.async_copy(bufs[b], out_hbm.at[pl.ds(base + c * _CH, _CH)],
                             wsems[b])

        def body(c, carry):
            for b in range(4):
                @pl.when(c % 4 == b)
                def _(b=b):
                    step(c, b)

            return carry

        lax.fori_loop(0, _NCH, body, 0)
        for b in range(4):
            cb = _NCH - 4 + ((b - _NCH) % 4)
            pltpu.make_async_copy(
                bufs[b], out_hbm.at[pl.ds(base + cb * _CH, _CH)], wsems[b]
            ).wait()

    return k(node_tokens, tgt3)


def _tc_logits(edge_tokens, tails, st_row, en_row, q, lng_col, lnb_row, W1,
               b1_row, W2, b2_2d):
    def body(et_ref, tl_ref, st_ref, en_ref, q_ref, lng_ref, lnb_ref, b1_ref,
             w1_ref, w2_ref, b2_ref, out_ref):
        f32 = jnp.float32
        e = et_ref[...]
        t = tl_ref[...]
        w1 = w1_ref[...]
        w1g = w1 * lng_ref[...]
        wbar = jnp.sum(w1g, axis=0, keepdims=True)
        hb = jnp.dot(lnb_ref[...], w1, preferred_element_type=f32) + b1_ref[...]
        w1e = w1g[:_HID]
        w1q = w1g[_HID:2 * _HID]
        w1t = w1g[2 * _HID:]
        qt = q_ref[...]
        qp = jnp.dot(qt, w1q, preferred_element_type=f32)
        sq = jnp.sum(qt, axis=1, keepdims=True)
        sqq = jnp.sum(qt * qt, axis=1, keepdims=True)
        eid = pl.program_id(0) * _EB + lax.broadcasted_iota(
            jnp.int32, (_EB, 1), 0
        )
        onehot = ((eid >= st_ref[...]) & (eid < en_ref[...])).astype(f32)
        qpe = jnp.dot(onehot, qp, preferred_element_type=f32)
        sqe = jnp.dot(onehot, sq, preferred_element_type=f32)
        sqqe = jnp.dot(onehot, sqq, preferred_element_type=f32)
        xw = (jnp.dot(e, w1e, preferred_element_type=f32)
              + jnp.dot(t, w1t, preferred_element_type=f32) + qpe)
        ones_col = jnp.ones((_HID, 1), f32)
        u = e + t
        v = e * e + t * t
        suv = jnp.dot(u, ones_col, preferred_element_type=f32)
        svv = jnp.dot(v, ones_col, preferred_element_type=f32)
        mu = (suv + sqe) * (1.0 / 384.0)
        var = (svv + sqqe) * (1.0 / 384.0) - mu * mu
        r = lax.rsqrt(var + 1e-5)
        h = xw * r - (mu * r) * wbar + hb
        gel = 0.5 * h * (1.0 + lax.erf(h * 0.7071067811865476))
        lg = jnp.dot(gel, w2_ref[...], preferred_element_type=f32) \
            + b2_ref[...]
        out_ref[...] = lg

    return pl.pallas_call(
        body,
        grid=(_NB,),
        in_specs=[
            pl.BlockSpec((_EB, _HID), lambda i: (i, 0)),
            pl.BlockSpec((_EB, _HID), lambda i: (i, 0)),
            pl.BlockSpec((1, _NG), lambda i: (0, 0)),
            pl.BlockSpec((1, _NG), lambda i: (0, 0)),
            pl.BlockSpec((_NG, _HID), lambda i: (0, 0)),
            pl.BlockSpec((3 * _HID, 1), lambda i: (0, 0)),
            pl.BlockSpec((1, 3 * _HID), lambda i: (0, 0)),
            pl.BlockSpec((1, _HID), lambda i: (0, 0)),
            pl.BlockSpec((3 * _HID, _HID), lambda i: (0, 0)),
            pl.BlockSpec((_HID, 1), lambda i: (0, 0)),
            pl.BlockSpec((1, 1), lambda i: (0, 0)),
        ],
        out_specs=pl.BlockSpec((_EB, 1), lambda i: (i, 0)),
        out_shape=jax.ShapeDtypeStruct((_N_EDGES, 1), jnp.float32),
    )(edge_tokens, tails, st_row, en_row, q, lng_col, lnb_row, b1_row, W1,
      W2, b2_2d)


_CHS = 80
_NCHS = _EPT // _CHS


def _sc_denom(lg2, tgt3):
    @functools.partial(
        pl.kernel,
        mesh=_sc_mesh(),
        out_type=(
            jax.ShapeDtypeStruct((2, _NPAD), jnp.float32),
            jax.ShapeDtypeStruct((2, 16), jnp.float32),
        ),
        scratch_types=[
            pltpu.VMEM((_EPT,), jnp.float32),
            pltpu.VMEM((_NCHS, _CHS), jnp.int32),
            pltpu.VMEM((16,), jnp.float32),
            pltpu.VMEM((_NPAD,), jnp.float32),
            pltpu.VMEM_SHARED((_NPAD,), jnp.float32),
            pltpu.VMEM_SHARED((256,), jnp.float32),
        ],
    )
    def k(lg_hbm, tgt_hbm, out_hbm, gm_hbm, lg_v, ix_v, gm_v, z_v, acc_sh,
          max_sh):
        wid = _wid()
        core = lax.axis_index("c")
        sid = lax.axis_index("s")
        pltpu.sync_copy(lg_hbm.at[wid], lg_v)
        pltpu.sync_copy(tgt_hbm.at[wid], ix_v)

        gm_v[...] = jnp.full((16,), -1e30, jnp.float32)

        def mb(j, c):
            gm_v[...] = jnp.maximum(gm_v[...], lg_v[pl.ds(j * 16, 16)])
            return c

        lax.fori_loop(0, _EPT // 16, mb, 0)
        pltpu.sync_copy(gm_v, max_sh.at[pl.ds(sid * 16, 16)])

        @pl.when(sid == 0)
        def _():
            zero = jnp.zeros((16,), jnp.float32)

            def zb(i, c):
                z_v[pl.ds(i * 16, 16)] = zero
                return c

            lax.fori_loop(0, _NPAD // 16, zb, 0)
            pltpu.sync_copy(z_v, acc_sh)

        plsc.subcore_barrier()
        pltpu.sync_copy(max_sh, z_v.at[pl.ds(0, 256)])
        mm = z_v[pl.ds(0, 16)]
        for r in range(1, 16):
            mm = jnp.maximum(mm, z_v[pl.ds(r * 16, 16)])
        s = mm[0]
        for i in range(1, 16):
            s = jnp.maximum(s, mm[i])
        m = jnp.full((16,), s, jnp.float32)
        gm_v[...] = m

        def eb(j, c):
            v = lg_v[pl.ds(j * 16, 16)]
            lg_v[pl.ds(j * 16, 16)] = jnp.exp(v - m)
            return c

        lax.fori_loop(0, _EPT // 16, eb, 0)

        def sb(ci, c):
            pltpu.sync_copy(
                lg_v.at[pl.ds(ci * _CHS, _CHS)],
                acc_sh.at[ix_v.at[ci]],
                add=True,
            )
            return c

        lax.fori_loop(0, _NCHS, sb, 0)
        plsc.subcore_barrier()

        @pl.when(sid == 0)
        def _():
            pltpu.sync_copy(acc_sh, out_hbm.at[core])
            pltpu.sync_copy(gm_v, gm_hbm.at[core])

    return k(lg2, tgt3)


def _tc_nodeoff(parts, gmax2):
    def body(p_ref, m_ref, o_ref):
        m0 = jnp.max(m_ref[0:1, :])
        m1 = jnp.max(m_ref[1:2, :])
        M = jnp.maximum(m0, m1)
        den = (p_ref[0:1, :] * jnp.exp(m0 - M)
               + p_ref[1:2, :] * jnp.exp(m1 - M))
        o_ref[...] = M + jnp.log(den)

    return pl.pallas_call(
        body, out_shape=jax.ShapeDtypeStruct((1, _NPAD), jnp.float32)
    )(parts, gmax2)


def _sc_graphsum(nodeoff, tgt3, lg2, eb3, sel3):
    @functools.partial(
        pl.kernel,
        mesh=_sc_mesh(),
        out_type=jax.ShapeDtypeStruct((2, 512), jnp.float32),
        scratch_types=[
            pltpu.VMEM((_NCHS, _CHS), jnp.int32),
            pltpu.VMEM_SHARED((_NPAD,), jnp.float32),
            pltpu.VMEM((_EPT,), jnp.float32),
            pltpu.VMEM((_EPT,), jnp.float32),
            pltpu.VMEM((_NCHS, _CHS), jnp.float32),
            pltpu.VMEM((_NCHS, _CHS), jnp.float32),
            pltpu.VMEM((_NCHS, _CHS), jnp.int32),
            pltpu.VMEM((_NCHS, _CHS), jnp.int32),
            pltpu.VMEM((512,), jnp.float32),
            pltpu.VMEM_SHARED((512,), jnp.float32),
        ],
    )
    def k(off_hbm, tgt_hbm, lg_hbm, eb_hbm, sel_hbm, out_hbm,
          idx_v, off_sh, ofs_v, lg_v, lps_v, sel_v, kv_v, kv2_v, z_v, acc_sh):
        wid = _wid()
        core = lax.axis_index("c")
        sid = lax.axis_index("s")
        pltpu.sync_copy(tgt_hbm.at[wid], idx_v)
        pltpu.sync_copy(lg_hbm.at[wid], lg_v)
        pltpu.sync_copy(eb_hbm.at[wid], kv_v)
        pltpu.sync_copy(sel_hbm.at[wid], sel_v)

        @pl.when(sid == 0)
        def _():
            pltpu.sync_copy(off_hbm, off_sh)
            zero = jnp.zeros((16,), jnp.float32)

            def zb(i, c):
                z_v[pl.ds(i * 16, 16)] = zero
                return c

            lax.fori_loop(0, 512 // 16, zb, 0)
            pltpu.sync_copy(z_v, acc_sh)

        plsc.subcore_barrier()

        def gb(ci, c):
            pltpu.sync_copy(
                off_sh.at[idx_v.at[ci]], ofs_v.at[pl.ds(ci * _CHS, _CHS)]
            )
            return c

        lax.fori_loop(0, _NCHS, gb, 0)

        kbase = sid * 16

        def cb(j, c):
            ci = j // (_CHS // 16)
            l = (j % (_CHS // 16)) * 16
            v = lg_v[pl.ds(j * 16, 16)]
            o = ofs_v[pl.ds(j * 16, 16)]
            s = sel_v[ci, pl.ds(l, 16)]
            g = kv_v[ci, pl.ds(l, 16)]
            lps_v[ci, pl.ds(l, 16)] = (v - o) * s
            kv_v[ci, pl.ds(l, 16)] = g + kbase
            kv2_v[ci, pl.ds(l, 16)] = g + (kbase + 256)
            return c

        lax.fori_loop(0, _EPT // 16, cb, 0)

        def sb(ci, c):
            pltpu.sync_copy(lps_v.at[ci], acc_sh.at[kv_v.at[ci]], add=True)
            pltpu.sync_copy(sel_v.at[ci], acc_sh.at[kv2_v.at[ci]], add=True)
            return c

        lax.fori_loop(0, _NCHS, sb, 0)
        plsc.subcore_barrier()

        @pl.when(sid == 0)
        def _():
            pltpu.sync_copy(acc_sh, out_hbm.at[core])

    return k(nodeoff, tgt3, lg2, eb3, sel3)


def _tc_final(parts):
    def body(p_ref, o1_ref, o2_ref):
        f32 = jnp.float32
        x = p_ref[...]
        tot = (jnp.sum(x[0:16], axis=0, keepdims=True)
               + jnp.sum(x[32:48], axis=0, keepdims=True))
        cnt = (jnp.sum(x[16:32], axis=0, keepdims=True)
               + jnp.sum(x[48:64], axis=0, keepdims=True))
        has = cnt > 0.0
        nll = jnp.where(has, -tot, 0.0)
        ngr = jnp.maximum(jnp.sum(has.astype(f32)), 1.0)
        o1_ref[...] = tot
        o2_ref[...] = (jnp.sum(nll) / ngr).reshape(1, 1)

    return pl.pallas_call(
        body,
        out_shape=[
            jax.ShapeDtypeStruct((1, _NG), jnp.float32),
            jax.ShapeDtypeStruct((1, 1), jnp.float32),
        ],
    )(parts)


def kernel(edge_tokens, node_tokens, question_tokens, ln_g, ln_b, W1, b1, W2,
           b2, edge_batch, selected_mask, edge_index):
    tgt = edge_index[1]
    tails = _sc_gather_tails(node_tokens, tgt.reshape(_NW, _NCH, _CH))
    starts = jnp.searchsorted(
        edge_batch, jnp.arange(_NG, dtype=edge_batch.dtype)
    ).astype(jnp.int32)
    ends = jnp.concatenate(
        [starts[1:], jnp.full((1,), _N_EDGES, jnp.int32)]
    )
    logits = _tc_logits(
        edge_tokens, tails,
        starts.reshape(1, _NG),
        ends.reshape(1, _NG),
        question_tokens,
        ln_g.reshape(3 * _HID, 1),
        ln_b.reshape(1, 3 * _HID),
        W1,
        b1.reshape(1, _HID),
        W2,
        b2.reshape(1, 1),
    )
    lg2 = logits.reshape(_NW, _EPT)
    tgt3 = tgt.reshape(_NW, _NCHS, _CHS)
    denp, gmax2 = _sc_denom(lg2, tgt3)
    nodeoff = _tc_nodeoff(denp, gmax2)
    parts = _sc_graphsum(
        nodeoff.reshape(_NPAD), tgt3, lg2,
        edge_batch.reshape(_NW, _NCHS, _CHS),
        selected_mask.astype(jnp.float32).reshape(_NW, _NCHS, _CHS),
    )
    logpb, nll = _tc_final(parts.reshape(64, _NG))
    return logpb.reshape(_NG), nll.reshape(())


# speedup vs baseline: 14.2681x; 1.0403x over previous
import functools

import jax
import jax.numpy as jnp
from jax import lax
from jax.experimental import pallas as pl
from jax.experimental.pallas import tpu as pltpu
from jax.experimental.pallas import tpu_sc as plsc

_N_NODES = 10000
_N_EDGES = 320000
_HID = 128
_NG = 16

_NW = 32
_EPT = _N_EDGES // _NW
_CH = 80
_NCH = _EPT // _CH
_NPAD = 10240
_EB = 4000
_NB = _N_EDGES // _EB


def _sc_mesh():
    return plsc.VectorSubcoreMesh(
        core_axis_name="c", subcore_axis_name="s", num_cores=2, num_subcores=16
    )


def _wid():
    return lax.axis_index("s") * 2 + lax.axis_index("c")


def _sc_gather_tails(node_tokens, tgt3):
    @functools.partial(
        pl.kernel,
        mesh=_sc_mesh(),
        out_type=jax.ShapeDtypeStruct((_N_EDGES, _HID), jnp.float32),
        scratch_types=[
            pltpu.VMEM((_NCH, _CH), jnp.int32),
            pltpu.VMEM((_CH, _HID), jnp.float32),
            pltpu.VMEM((_CH, _HID), jnp.float32),
            pltpu.VMEM((_CH, _HID), jnp.float32),
            pltpu.VMEM((_CH, _HID), jnp.float32),
            pltpu.SemaphoreType.DMA,
            pltpu.SemaphoreType.DMA,
            pltpu.SemaphoreType.DMA,
            pltpu.SemaphoreType.DMA,
            pltpu.SemaphoreType.DMA,
            pltpu.SemaphoreType.DMA,
            pltpu.SemaphoreType.DMA,
            pltpu.SemaphoreType.DMA,
        ],
    )
    def k(node_hbm, tgt_hbm, out_hbm, idx_v, b0, b1, b2, b3,
          g0, g1, g2, g3, w0, w1, w2, w3):
        wid = _wid()
        base = wid * _EPT
        bufs = (b0, b1, b2, b3)
        gsems = (g0, g1, g2, g3)
        wsems = (w0, w1, w2, w3)
        pltpu.sync_copy(tgt_hbm.at[wid], idx_v)
        pltpu.async_copy(node_hbm.at[idx_v.at[0]], b0, g0)
        pltpu.async_copy(node_hbm.at[idx_v.at[1]], b1, g1)

        def step(c, b):
            bp = (b + 2) % 4

            @pl.when(c + 2 < _NCH)
            def _():
                @pl.when(c >= 2)
                def _():
                    pltpu.make_async_copy(
                        bufs[bp],
                        out_hbm.at[pl.ds(base + (c - 2) * _CH, _CH)],
                        wsems[bp],
                    ).wait()

                pltpu.async_copy(node_hbm.at[idx_v.at[c + 2]], bufs[bp],
                                 gsems[bp])

            pltpu.make_async_copy(node_hbm.at[idx_v.at[c]], bufs[b],
                                  gsems[b]).wait()
            pltpu.async_copy(bufs[b], out_hbm.at[pl.ds(base + c * _CH, _CH)],
                             wsems[b])

        def body(c, carry):
            for b in range(4):
                @pl.when(c % 4 == b)
                def _(b=b):
                    step(c, b)

            return carry

        lax.fori_loop(0, _NCH, body, 0)
        for b in range(4):
            cb = _NCH - 4 + ((b - _NCH) % 4)
            pltpu.make_async_copy(
                bufs[b], out_hbm.at[pl.ds(base + cb * _CH, _CH)], wsems[b]
            ).wait()

    return k(node_tokens, tgt3)


def _tc_logits(edge_tokens, tails, st_row, en_row, q, lng_col, lnb_row, W1,
               b1_row, W2, b2_2d):
    def body(et_ref, tl_ref, st_ref, en_ref, q_ref, lng_ref, lnb_ref, b1_ref,
             w1_ref, w2_ref, b2_ref, out_ref):
        f32 = jnp.float32
        e = et_ref[...]
        t = tl_ref[...]
        w1 = w1_ref[...]
        w1g = w1 * lng_ref[...]
        wbar = jnp.sum(w1g, axis=0, keepdims=True)
        hb = jnp.dot(lnb_ref[...], w1, preferred_element_type=f32) + b1_ref[...]
        w1e = w1g[:_HID]
        w1q = w1g[_HID:2 * _HID]
        w1t = w1g[2 * _HID:]
        qt = q_ref[...]
        qp = jnp.dot(qt, w1q, preferred_element_type=f32)
        sq = jnp.sum(qt, axis=1, keepdims=True)
        sqq = jnp.sum(qt * qt, axis=1, keepdims=True)
        eid = pl.program_id(0) * _EB + lax.broadcasted_iota(
            jnp.int32, (_EB, 1), 0
        )
        onehot = ((eid >= st_ref[...]) & (eid < en_ref[...])).astype(f32)
        qpe = jnp.dot(onehot, qp, preferred_element_type=f32)
        sqe = jnp.dot(onehot, sq, preferred_element_type=f32)
        sqqe = jnp.dot(onehot, sqq, preferred_element_type=f32)
        xw = (jnp.dot(e, w1e, preferred_element_type=f32)
              + jnp.dot(t, w1t, preferred_element_type=f32) + qpe)
        ones_col = jnp.ones((_HID, 1), f32)
        u = e + t
        v = e * e + t * t
        suv = jnp.dot(u, ones_col, preferred_element_type=f32)
        svv = jnp.dot(v, ones_col, preferred_element_type=f32)
        mu = (suv + sqe) * (1.0 / 384.0)
        var = (svv + sqqe) * (1.0 / 384.0) - mu * mu
        r = lax.rsqrt(var + 1e-5)
        h = xw * r - (mu * r) * wbar + hb
        gel = 0.5 * h * (1.0 + lax.erf(h * 0.7071067811865476))
        lg = jnp.dot(gel, w2_ref[...], preferred_element_type=f32) \
            + b2_ref[...]
        out_ref[...] = lg

    return pl.pallas_call(
        body,
        grid=(_NB,),
        in_specs=[
            pl.BlockSpec((_EB, _HID), lambda i: (i, 0)),
            pl.BlockSpec((_EB, _HID), lambda i: (i, 0)),
            pl.BlockSpec((1, _NG), lambda i: (0, 0)),
            pl.BlockSpec((1, _NG), lambda i: (0, 0)),
            pl.BlockSpec((_NG, _HID), lambda i: (0, 0)),
            pl.BlockSpec((3 * _HID, 1), lambda i: (0, 0)),
            pl.BlockSpec((1, 3 * _HID), lambda i: (0, 0)),
            pl.BlockSpec((1, _HID), lambda i: (0, 0)),
            pl.BlockSpec((3 * _HID, _HID), lambda i: (0, 0)),
            pl.BlockSpec((_HID, 1), lambda i: (0, 0)),
            pl.BlockSpec((1, 1), lambda i: (0, 0)),
        ],
        out_specs=pl.BlockSpec((_EB, 1), lambda i: (i, 0)),
        out_shape=jax.ShapeDtypeStruct((_N_EDGES, 1), jnp.float32),
    )(edge_tokens, tails, st_row, en_row, q, lng_col, lnb_row, b1_row, W1,
      W2, b2_2d)


_CHS = 80
_NCHS = _EPT // _CHS


def _sc_denom(lg2, tgt3):
    @functools.partial(
        pl.kernel,
        mesh=_sc_mesh(),
        out_type=(
            jax.ShapeDtypeStruct((2, _NPAD), jnp.float32),
            jax.ShapeDtypeStruct((2, 16), jnp.float32),
        ),
        scratch_types=[
            pltpu.VMEM((_EPT,), jnp.float32),
            pltpu.VMEM((_NCHS, _CHS), jnp.int32),
            pltpu.VMEM((16,), jnp.float32),
            pltpu.VMEM((_NPAD,), jnp.float32),
            pltpu.VMEM_SHARED((_NPAD,), jnp.float32),
            pltpu.VMEM_SHARED((256,), jnp.float32),
        ],
    )
    def k(lg_hbm, tgt_hbm, out_hbm, gm_hbm, lg_v, ix_v, gm_v, z_v, acc_sh,
          max_sh):
        wid = _wid()
        core = lax.axis_index("c")
        sid = lax.axis_index("s")
        pltpu.sync_copy(lg_hbm.at[wid], lg_v)
        pltpu.sync_copy(tgt_hbm.at[wid], ix_v)

        gm_v[...] = jnp.full((16,), -1e30, jnp.float32)

        def mb(j, c):
            gm_v[...] = jnp.maximum(gm_v[...], lg_v[pl.ds(j * 16, 16)])
            return c

        lax.fori_loop(0, _EPT // 16, mb, 0)
        pltpu.sync_copy(gm_v, max_sh.at[pl.ds(sid * 16, 16)])

        @pl.when(sid == 0)
        def _():
            zero = jnp.zeros((16,), jnp.float32)

            def zb(i, c):
                z_v[pl.ds(i * 16, 16)] = zero
                return c

            lax.fori_loop(0, _NPAD // 16, zb, 0)
            pltpu.sync_copy(z_v, acc_sh)

        plsc.subcore_barrier()
        pltpu.sync_copy(max_sh, z_v.at[pl.ds(0, 256)])
        mm = z_v[pl.ds(0, 16)]
        for r in range(1, 16):
            mm = jnp.maximum(mm, z_v[pl.ds(r * 16, 16)])
        s = mm[0]
        for i in range(1, 16):
            s = jnp.maximum(s, mm[i])
        m = jnp.full((16,), s, jnp.float32)
        gm_v[...] = m

        def eb(j, c):
            v = lg_v[pl.ds(j * 16, 16)]
            lg_v[pl.ds(j * 16, 16)] = jnp.exp(v - m)
            return c

        lax.fori_loop(0, _EPT // 16, eb, 0)

        def sb(ci, c):
            pltpu.sync_copy(
                lg_v.at[pl.ds(ci * _CHS, _CHS)],
                acc_sh.at[ix_v.at[ci]],
                add=True,
            )
            return c

        lax.fori_loop(0, _NCHS, sb, 0)
        plsc.subcore_barrier()

        @pl.when(sid == 0)
        def _():
            pltpu.sync_copy(acc_sh, out_hbm.at[core])
            pltpu.sync_copy(gm_v, gm_hbm.at[core])

    return k(lg2, tgt3)


def _tc_nodeoff(parts, gmax2):
    def body(p_ref, m_ref, o_ref):
        m0 = jnp.max(m_ref[0:1, :])
        m1 = jnp.max(m_ref[1:2, :])
        M = jnp.maximum(m0, m1)
        den = (p_ref[0:1, :] * jnp.exp(m0 - M)
               + p_ref[1:2, :] * jnp.exp(m1 - M))
        o_ref[...] = M + jnp.log(den)

    return pl.pallas_call(
        body, out_shape=jax.ShapeDtypeStruct((1, _NPAD), jnp.float32)
    )(parts, gmax2)


def _sc_graphsum(nodeoff, tgt3, lg2, eb3, sel3):
    @functools.partial(
        pl.kernel,
        mesh=_sc_mesh(),
        out_type=jax.ShapeDtypeStruct((2, 512), jnp.float32),
        scratch_types=[
            pltpu.VMEM((_NCHS, _CHS), jnp.int32),
            pltpu.VMEM_SHARED((_NPAD,), jnp.float32),
            pltpu.VMEM((_EPT,), jnp.float32),
            pltpu.VMEM((_EPT,), jnp.float32),
            pltpu.VMEM((_NCHS, _CHS), jnp.float32),
            pltpu.VMEM((_NCHS, _CHS), jnp.float32),
            pltpu.VMEM((_NCHS, _CHS), jnp.int32),
            pltpu.VMEM((_NCHS, _CHS), jnp.int32),
            pltpu.VMEM((512,), jnp.float32),
            pltpu.VMEM_SHARED((512,), jnp.float32),
        ],
    )
    def k(off_hbm, tgt_hbm, lg_hbm, eb_hbm, sel_hbm, out_hbm,
          idx_v, off_sh, ofs_v, lg_v, lps_v, sel_v, kv_v, kv2_v, z_v, acc_sh):
        wid = _wid()
        core = lax.axis_index("c")
        sid = lax.axis_index("s")
        pltpu.sync_copy(tgt_hbm.at[wid], idx_v)
        pltpu.sync_copy(lg_hbm.at[wid], lg_v)
        pltpu.sync_copy(eb_hbm.at[wid], kv_v)
        pltpu.sync_copy(sel_hbm.at[wid], sel_v)

        @pl.when(sid == 0)
        def _():
            pltpu.sync_copy(off_hbm, off_sh)
            zero = jnp.zeros((16,), jnp.float32)

            def zb(i, c):
                z_v[pl.ds(i * 16, 16)] = zero
                return c

            lax.fori_loop(0, 512 // 16, zb, 0)
            pltpu.sync_copy(z_v, acc_sh)

        plsc.subcore_barrier()

        def gb(ci, c):
            pltpu.sync_copy(
                off_sh.at[idx_v.at[ci]], ofs_v.at[pl.ds(ci * _CHS, _CHS)]
            )
            return c

        lax.fori_loop(0, _NCHS, gb, 0)

        kbase = sid * 16

        def cb(j, c):
            ci = j // (_CHS // 16)
            l = (j % (_CHS // 16)) * 16
            v = lg_v[pl.ds(j * 16, 16)]
            o = ofs_v[pl.ds(j * 16, 16)]
            s = sel_v[ci, pl.ds(l, 16)]
            g = kv_v[ci, pl.ds(l, 16)]
            lps_v[ci, pl.ds(l, 16)] = (v - o) * s
            kv_v[ci, pl.ds(l, 16)] = g + kbase
            kv2_v[ci, pl.ds(l, 16)] = g + (kbase + 256)
            return c

        lax.fori_loop(0, _EPT // 16, cb, 0)

        def sb(ci, c):
            pltpu.sync_copy(lps_v.at[ci], acc_sh.at[kv_v.at[ci]], add=True)
            pltpu.sync_copy(sel_v.at[ci], acc_sh.at[kv2_v.at[ci]], add=True)
            return c

        lax.fori_loop(0, _NCHS, sb, 0)
        plsc.subcore_barrier()

        @pl.when(sid == 0)
        def _():
            pltpu.sync_copy(acc_sh, out_hbm.at[core])

    return k(nodeoff, tgt3, lg2, eb3, sel3)


def _tc_final(parts):
    def body(p_ref, o1_ref, o2_ref):
        f32 = jnp.float32
        x = p_ref[...]
        tot = (jnp.sum(x[0:16], axis=0, keepdims=True)
               + jnp.sum(x[32:48], axis=0, keepdims=True))
        cnt = (jnp.sum(x[16:32], axis=0, keepdims=True)
               + jnp.sum(x[48:64], axis=0, keepdims=True))
        has = cnt > 0.0
        nll = jnp.where(has, -tot, 0.0)
        ngr = jnp.maximum(jnp.sum(has.astype(f32)), 1.0)
        o1_ref[...] = tot
        o2_ref[...] = (jnp.sum(nll) / ngr).reshape(1, 1)

    return pl.pallas_call(
        body,
        out_shape=[
            jax.ShapeDtypeStruct((1, _NG), jnp.float32),
            jax.ShapeDtypeStruct((1, 1), jnp.float32),
        ],
    )(parts)


def kernel(edge_tokens, node_tokens, question_tokens, ln_g, ln_b, W1, b1, W2,
           b2, edge_batch, selected_mask, edge_index):
    tgt = edge_index[1]
    tails = _sc_gather_tails(node_tokens, tgt.reshape(_NW, _NCH, _CH))
    starts = jnp.searchsorted(
        edge_batch, jnp.arange(_NG, dtype=edge_batch.dtype)
    ).astype(jnp.int32)
    ends = jnp.concatenate(
        [starts[1:], jnp.full((1,), _N_EDGES, jnp.int32)]
    )
    logits = _tc_logits(
        edge_tokens, tails,
        starts.reshape(1, _NG),
        ends.reshape(1, _NG),
        question_tokens,
        ln_g.reshape(3 * _HID, 1),
        ln_b.reshape(1, 3 * _HID),
        W1,
        b1.reshape(1, _HID),
        W2,
        b2.reshape(1, 1),
    )
    lg2 = logits.reshape(_NW, _EPT)
    tgt3 = tgt.reshape(_NW, _NCHS, _CHS)
    denp, gmax2 = _sc_denom(lg2, tgt3)
    nodeoff = _tc_nodeoff(denp, gmax2)
    parts = _sc_graphsum(
        nodeoff.reshape(_NPAD), tgt3, lg2,
        edge_batch.reshape(_NW, _NCHS, _CHS),
        selected_mask.astype(jnp.float32).reshape(_NW, _NCHS, _CHS),
    )
    logpb, nll = _tc_final(parts.reshape(64, _NG))
    return logpb.reshape(_NG), nll.reshape(())


# speedup vs baseline: 14.8273x; 1.0392x over previous
import functools

import jax
import jax.numpy as jnp
from jax import lax
from jax.experimental import pallas as pl
from jax.experimental.pallas import tpu as pltpu
from jax.experimental.pallas import tpu_sc as plsc

_N_NODES = 10000
_N_EDGES = 320000
_HID = 128
_NG = 16

_NW = 32
_EPT = _N_EDGES // _NW
_CH = 80
_NCH = _EPT // _CH
_NPAD = 10240
_EB = 8000
_NB = _N_EDGES // _EB


def _sc_mesh():
    return plsc.VectorSubcoreMesh(
        core_axis_name="c", subcore_axis_name="s", num_cores=2, num_subcores=16
    )


def _wid():
    return lax.axis_index("s") * 2 + lax.axis_index("c")


def _sc_gather_tails(node_tokens, tgt3):
    @functools.partial(
        pl.kernel,
        mesh=_sc_mesh(),
        out_type=jax.ShapeDtypeStruct((_N_EDGES, _HID), jnp.float32),
        scratch_types=[
            pltpu.VMEM((_NCH, _CH), jnp.int32),
            pltpu.VMEM((_CH, _HID), jnp.float32),
            pltpu.VMEM((_CH, _HID), jnp.float32),
            pltpu.VMEM((_CH, _HID), jnp.float32),
            pltpu.VMEM((_CH, _HID), jnp.float32),
            pltpu.SemaphoreType.DMA,
            pltpu.SemaphoreType.DMA,
            pltpu.SemaphoreType.DMA,
            pltpu.SemaphoreType.DMA,
            pltpu.SemaphoreType.DMA,
            pltpu.SemaphoreType.DMA,
            pltpu.SemaphoreType.DMA,
            pltpu.SemaphoreType.DMA,
        ],
    )
    def k(node_hbm, tgt_hbm, out_hbm, idx_v, b0, b1, b2, b3,
          g0, g1, g2, g3, w0, w1, w2, w3):
        wid = _wid()
        base = wid * _EPT
        bufs = (b0, b1, b2, b3)
        gsems = (g0, g1, g2, g3)
        wsems = (w0, w1, w2, w3)
        pltpu.sync_copy(tgt_hbm.at[wid], idx_v)
        pltpu.async_copy(node_hbm.at[idx_v.at[0]], b0, g0)
        pltpu.async_copy(node_hbm.at[idx_v.at[1]], b1, g1)

        def step(c, b):
            bp = (b + 2) % 4

            @pl.when(c + 2 < _NCH)
            def _():
                @pl.when(c >= 2)
                def _():
                    pltpu.make_async_copy(
                        bufs[bp],
                        out_hbm.at[pl.ds(base + (c - 2) * _CH, _CH)],
                        wsems[bp],
                    ).wait()

                pltpu.async_copy(node_hbm.at[idx_v.at[c + 2]], bufs[bp],
                                 gsems[bp])

            pltpu.make_async_copy(node_hbm.at[idx_v.at[c]], bufs[b],
                                  gsems[b]).wait()
            pltpu.async_copy(bufs[b], out_hbm.at[pl.ds(base + c * _CH, _CH)],
                             wsems[b])

        def body(c, carry):
            for b in range(4):
                @pl.when(c % 4 == b)
                def _(b=b):
                    step(c, b)

            return carry

        lax.fori_loop(0, _NCH, body, 0)
        for b in range(4):
            cb = _NCH - 4 + ((b - _NCH) % 4)
            pltpu.make_async_copy(
                bufs[b], out_hbm.at[pl.ds(base + cb * _CH, _CH)], wsems[b]
            ).wait()

    return k(node_tokens, tgt3)


def _tc_logits(edge_tokens, tails, st_row, en_row, q, lng_col, lnb_row, W1,
               b1_row, W2, b2_2d):
    def body(et_ref, tl_ref, st_ref, en_ref, q_ref, lng_ref, lnb_ref, b1_ref,
             w1_ref, w2_ref, b2_ref, out_ref):
        f32 = jnp.float32
        e = et_ref[...]
        t = tl_ref[...]
        w1 = w1_ref[...]
        w1g = w1 * lng_ref[...]
        wbar = jnp.sum(w1g, axis=0, keepdims=True)
        hb = jnp.dot(lnb_ref[...], w1, preferred_element_type=f32) + b1_ref[...]
        w1e = w1g[:_HID]
        w1q = w1g[_HID:2 * _HID]
        w1t = w1g[2 * _HID:]
        qt = q_ref[...]
        qp = jnp.dot(qt, w1q, preferred_element_type=f32)
        sq = jnp.sum(qt, axis=1, keepdims=True)
        sqq = jnp.sum(qt * qt, axis=1, keepdims=True)
        eid = pl.program_id(0) * _EB + lax.broadcasted_iota(
            jnp.int32, (_EB, 1), 0
        )
        onehot = ((eid >= st_ref[...]) & (eid < en_ref[...])).astype(f32)
        qpe = jnp.dot(onehot, qp, preferred_element_type=f32)
        sqe = jnp.dot(onehot, sq, preferred_element_type=f32)
        sqqe = jnp.dot(onehot, sqq, preferred_element_type=f32)
        xw = (jnp.dot(e, w1e, preferred_element_type=f32)
              + jnp.dot(t, w1t, preferred_element_type=f32) + qpe)
        ones_col = jnp.ones((_HID, 1), f32)
        u = e + t
        v = e * e + t * t
        suv = jnp.dot(u, ones_col, preferred_element_type=f32)
        svv = jnp.dot(v, ones_col, preferred_element_type=f32)
        mu = (suv + sqe) * (1.0 / 384.0)
        var = (svv + sqqe) * (1.0 / 384.0) - mu * mu
        r = lax.rsqrt(var + 1e-5)
        h = xw * r - (mu * r) * wbar + hb
        gel = 0.5 * h * (1.0 + lax.erf(h * 0.7071067811865476))
        lg = jnp.dot(gel, w2_ref[...], preferred_element_type=f32) \
            + b2_ref[...]
        out_ref[...] = lg

    return pl.pallas_call(
        body,
        grid=(_NB,),
        in_specs=[
            pl.BlockSpec((_EB, _HID), lambda i: (i, 0)),
            pl.BlockSpec((_EB, _HID), lambda i: (i, 0)),
            pl.BlockSpec((1, _NG), lambda i: (0, 0)),
            pl.BlockSpec((1, _NG), lambda i: (0, 0)),
            pl.BlockSpec((_NG, _HID), lambda i: (0, 0)),
            pl.BlockSpec((3 * _HID, 1), lambda i: (0, 0)),
            pl.BlockSpec((1, 3 * _HID), lambda i: (0, 0)),
            pl.BlockSpec((1, _HID), lambda i: (0, 0)),
            pl.BlockSpec((3 * _HID, _HID), lambda i: (0, 0)),
            pl.BlockSpec((_HID, 1), lambda i: (0, 0)),
            pl.BlockSpec((1, 1), lambda i: (0, 0)),
        ],
        out_specs=pl.BlockSpec((_EB, 1), lambda i: (i, 0)),
        out_shape=jax.ShapeDtypeStruct((_N_EDGES, 1), jnp.float32),
    )(edge_tokens, tails, st_row, en_row, q, lng_col, lnb_row, b1_row, W1,
      W2, b2_2d)


_CHS = 80
_NCHS = _EPT // _CHS


def _sc_denom(lg2, tgt3):
    @functools.partial(
        pl.kernel,
        mesh=_sc_mesh(),
        out_type=(
            jax.ShapeDtypeStruct((2, _NPAD), jnp.float32),
            jax.ShapeDtypeStruct((2, 16), jnp.float32),
        ),
        scratch_types=[
            pltpu.VMEM((_EPT,), jnp.float32),
            pltpu.VMEM((_NCHS, _CHS), jnp.int32),
            pltpu.VMEM((16,), jnp.float32),
            pltpu.VMEM((_NPAD,), jnp.float32),
            pltpu.VMEM_SHARED((_NPAD,), jnp.float32),
            pltpu.VMEM_SHARED((256,), jnp.float32),
        ],
    )
    def k(lg_hbm, tgt_hbm, out_hbm, gm_hbm, lg_v, ix_v, gm_v, z_v, acc_sh,
          max_sh):
        wid = _wid()
        core = lax.axis_index("c")
        sid = lax.axis_index("s")
        pltpu.sync_copy(lg_hbm.at[wid], lg_v)
        pltpu.sync_copy(tgt_hbm.at[wid], ix_v)

        gm_v[...] = jnp.full((16,), -1e30, jnp.float32)

        def mb(j, c):
            gm_v[...] = jnp.maximum(gm_v[...], lg_v[pl.ds(j * 16, 16)])
            return c

        lax.fori_loop(0, _EPT // 16, mb, 0)
        pltpu.sync_copy(gm_v, max_sh.at[pl.ds(sid * 16, 16)])

        @pl.when(sid == 0)
        def _():
            zero = jnp.zeros((16,), jnp.float32)

            def zb(i, c):
                z_v[pl.ds(i * 16, 16)] = zero
                return c

            lax.fori_loop(0, _NPAD // 16, zb, 0)
            pltpu.sync_copy(z_v, acc_sh)

        plsc.subcore_barrier()
        pltpu.sync_copy(max_sh, z_v.at[pl.ds(0, 256)])
        mm = z_v[pl.ds(0, 16)]
        for r in range(1, 16):
            mm = jnp.maximum(mm, z_v[pl.ds(r * 16, 16)])
        s = mm[0]
        for i in range(1, 16):
            s = jnp.maximum(s, mm[i])
        m = jnp.full((16,), s, jnp.float32)
        gm_v[...] = m

        def eb(j, c):
            v = lg_v[pl.ds(j * 16, 16)]
            lg_v[pl.ds(j * 16, 16)] = jnp.exp(v - m)
            return c

        lax.fori_loop(0, _EPT // 16, eb, 0)

        def sb(ci, c):
            pltpu.sync_copy(
                lg_v.at[pl.ds(ci * _CHS, _CHS)],
                acc_sh.at[ix_v.at[ci]],
                add=True,
            )
            return c

        lax.fori_loop(0, _NCHS, sb, 0)
        plsc.subcore_barrier()

        @pl.when(sid == 0)
        def _():
            pltpu.sync_copy(acc_sh, out_hbm.at[core])
            pltpu.sync_copy(gm_v, gm_hbm.at[core])

    return k(lg2, tgt3)


def _tc_nodeoff(parts, gmax2):
    def body(p_ref, m_ref, o_ref):
        m0 = jnp.max(m_ref[0:1, :])
        m1 = jnp.max(m_ref[1:2, :])
        M = jnp.maximum(m0, m1)
        den = (p_ref[0:1, :] * jnp.exp(m0 - M)
               + p_ref[1:2, :] * jnp.exp(m1 - M))
        o_ref[...] = M + jnp.log(den)

    return pl.pallas_call(
        body, out_shape=jax.ShapeDtypeStruct((1, _NPAD), jnp.float32)
    )(parts, gmax2)


def _sc_graphsum(nodeoff, tgt3, lg2, eb3, sel3):
    @functools.partial(
        pl.kernel,
        mesh=_sc_mesh(),
        out_type=jax.ShapeDtypeStruct((2, 512), jnp.float32),
        scratch_types=[
            pltpu.VMEM((_NCHS, _CHS), jnp.int32),
            pltpu.VMEM_SHARED((_NPAD,), jnp.float32),
            pltpu.VMEM((_EPT,), jnp.float32),
            pltpu.VMEM((_EPT,), jnp.float32),
            pltpu.VMEM((_NCHS, _CHS), jnp.float32),
            pltpu.VMEM((_NCHS, _CHS), jnp.float32),
            pltpu.VMEM((_NCHS, _CHS), jnp.int32),
            pltpu.VMEM((_NCHS, _CHS), jnp.int32),
            pltpu.VMEM((512,), jnp.float32),
            pltpu.VMEM_SHARED((512,), jnp.float32),
        ],
    )
    def k(off_hbm, tgt_hbm, lg_hbm, eb_hbm, sel_hbm, out_hbm,
          idx_v, off_sh, ofs_v, lg_v, lps_v, sel_v, kv_v, kv2_v, z_v, acc_sh):
        wid = _wid()
        core = lax.axis_index("c")
        sid = lax.axis_index("s")
        pltpu.sync_copy(tgt_hbm.at[wid], idx_v)
        pltpu.sync_copy(lg_hbm.at[wid], lg_v)
        pltpu.sync_copy(eb_hbm.at[wid], kv_v)
        pltpu.sync_copy(sel_hbm.at[wid], sel_v)

        @pl.when(sid == 0)
        def _():
            pltpu.sync_copy(off_hbm, off_sh)
            zero = jnp.zeros((16,), jnp.float32)

            def zb(i, c):
                z_v[pl.ds(i * 16, 16)] = zero
                return c

            lax.fori_loop(0, 512 // 16, zb, 0)
            pltpu.sync_copy(z_v, acc_sh)

        plsc.subcore_barrier()

        def gb(ci, c):
            pltpu.sync_copy(
                off_sh.at[idx_v.at[ci]], ofs_v.at[pl.ds(ci * _CHS, _CHS)]
            )
            return c

        lax.fori_loop(0, _NCHS, gb, 0)

        kbase = sid * 16

        def cb(j, c):
            ci = j // (_CHS // 16)
            l = (j % (_CHS // 16)) * 16
            v = lg_v[pl.ds(j * 16, 16)]
            o = ofs_v[pl.ds(j * 16, 16)]
            s = sel_v[ci, pl.ds(l, 16)]
            g = kv_v[ci, pl.ds(l, 16)]
            lps_v[ci, pl.ds(l, 16)] = (v - o) * s
            kv_v[ci, pl.ds(l, 16)] = g + kbase
            kv2_v[ci, pl.ds(l, 16)] = g + (kbase + 256)
            return c

        lax.fori_loop(0, _EPT // 16, cb, 0)

        def sb(ci, c):
            pltpu.sync_copy(lps_v.at[ci], acc_sh.at[kv_v.at[ci]], add=True)
            pltpu.sync_copy(sel_v.at[ci], acc_sh.at[kv2_v.at[ci]], add=True)
            return c

        lax.fori_loop(0, _NCHS, sb, 0)
        plsc.subcore_barrier()

        @pl.when(sid == 0)
        def _():
            pltpu.sync_copy(acc_sh, out_hbm.at[core])

    return k(nodeoff, tgt3, lg2, eb3, sel3)


def _tc_final(parts):
    def body(p_ref, o1_ref, o2_ref):
        f32 = jnp.float32
        x = p_ref[...]
        tot = (jnp.sum(x[0:16], axis=0, keepdims=True)
               + jnp.sum(x[32:48], axis=0, keepdims=True))
        cnt = (jnp.sum(x[16:32], axis=0, keepdims=True)
               + jnp.sum(x[48:64], axis=0, keepdims=True))
        has = cnt > 0.0
        nll = jnp.where(has, -tot, 0.0)
        ngr = jnp.maximum(jnp.sum(has.astype(f32)), 1.0)
        o1_ref[...] = tot
        o2_ref[...] = (jnp.sum(nll) / ngr).reshape(1, 1)

    return pl.pallas_call(
        body,
        out_shape=[
            jax.ShapeDtypeStruct((1, _NG), jnp.float32),
            jax.ShapeDtypeStruct((1, 1), jnp.float32),
        ],
    )(parts)


def kernel(edge_tokens, node_tokens, question_tokens, ln_g, ln_b, W1, b1, W2,
           b2, edge_batch, selected_mask, edge_index):
    tgt = edge_index[1]
    tails = _sc_gather_tails(node_tokens, tgt.reshape(_NW, _NCH, _CH))
    starts = jnp.searchsorted(
        edge_batch, jnp.arange(_NG, dtype=edge_batch.dtype)
    ).astype(jnp.int32)
    ends = jnp.concatenate(
        [starts[1:], jnp.full((1,), _N_EDGES, jnp.int32)]
    )
    logits = _tc_logits(
        edge_tokens, tails,
        starts.reshape(1, _NG),
        ends.reshape(1, _NG),
        question_tokens,
        ln_g.reshape(3 * _HID, 1),
        ln_b.reshape(1, 3 * _HID),
        W1,
        b1.reshape(1, _HID),
        W2,
        b2.reshape(1, 1),
    )
    lg2 = logits.reshape(_NW, _EPT)
    tgt3 = tgt.reshape(_NW, _NCHS, _CHS)
    denp, gmax2 = _sc_denom(lg2, tgt3)
    nodeoff = _tc_nodeoff(denp, gmax2)
    parts = _sc_graphsum(
        nodeoff.reshape(_NPAD), tgt3, lg2,
        edge_batch.reshape(_NW, _NCHS, _CHS),
        selected_mask.astype(jnp.float32).reshape(_NW, _NCHS, _CHS),
    )
    logpb, nll = _tc_final(parts.reshape(64, _NG))
    return logpb.reshape(_NG), nll.reshape(())


# speedup vs baseline: 14.8407x; 1.0009x over previous
import functools

import jax
import jax.numpy as jnp
from jax import lax
from jax.experimental import pallas as pl
from jax.experimental.pallas import tpu as pltpu
from jax.experimental.pallas import tpu_sc as plsc

_N_NODES = 10000
_N_EDGES = 320000
_HID = 128
_NG = 16

_NW = 32
_EPT = _N_EDGES // _NW
_CH = 80
_NCH = _EPT // _CH
_NPAD = 10240
_EB = 10000
_NB = _N_EDGES // _EB


def _sc_mesh():
    return plsc.VectorSubcoreMesh(
        core_axis_name="c", subcore_axis_name="s", num_cores=2, num_subcores=16
    )


def _wid():
    return lax.axis_index("s") * 2 + lax.axis_index("c")


def _sc_gather_tails(node_tokens, tgt3):
    @functools.partial(
        pl.kernel,
        mesh=_sc_mesh(),
        out_type=jax.ShapeDtypeStruct((_N_EDGES, _HID), jnp.float32),
        scratch_types=[
            pltpu.VMEM((_NCH, _CH), jnp.int32),
            pltpu.VMEM((_CH, _HID), jnp.float32),
            pltpu.VMEM((_CH, _HID), jnp.float32),
            pltpu.VMEM((_CH, _HID), jnp.float32),
            pltpu.VMEM((_CH, _HID), jnp.float32),
            pltpu.SemaphoreType.DMA,
            pltpu.SemaphoreType.DMA,
            pltpu.SemaphoreType.DMA,
            pltpu.SemaphoreType.DMA,
            pltpu.SemaphoreType.DMA,
            pltpu.SemaphoreType.DMA,
            pltpu.SemaphoreType.DMA,
            pltpu.SemaphoreType.DMA,
        ],
    )
    def k(node_hbm, tgt_hbm, out_hbm, idx_v, b0, b1, b2, b3,
          g0, g1, g2, g3, w0, w1, w2, w3):
        wid = _wid()
        base = wid * _EPT
        bufs = (b0, b1, b2, b3)
        gsems = (g0, g1, g2, g3)
        wsems = (w0, w1, w2, w3)
        pltpu.sync_copy(tgt_hbm.at[wid], idx_v)
        pltpu.async_copy(node_hbm.at[idx_v.at[0]], b0, g0)
        pltpu.async_copy(node_hbm.at[idx_v.at[1]], b1, g1)

        def step(c, b):
            bp = (b + 2) % 4

            @pl.when(c + 2 < _NCH)
            def _():
                @pl.when(c >= 2)
                def _():
                    pltpu.make_async_copy(
                        bufs[bp],
                        out_hbm.at[pl.ds(base + (c - 2) * _CH, _CH)],
                        wsems[bp],
                    ).wait()

                pltpu.async_copy(node_hbm.at[idx_v.at[c + 2]], bufs[bp],
                                 gsems[bp])

            pltpu.make_async_copy(node_hbm.at[idx_v.at[c]], bufs[b],
                                  gsems[b]).wait()
            pltpu.async_copy(bufs[b], out_hbm.at[pl.ds(base + c * _CH, _CH)],
                             wsems[b])

        def body(c, carry):
            for b in range(4):
                @pl.when(c % 4 == b)
                def _(b=b):
                    step(c, b)

            return carry

        lax.fori_loop(0, _NCH, body, 0)
        for b in range(4):
            cb = _NCH - 4 + ((b - _NCH) % 4)
            pltpu.make_async_copy(
                bufs[b], out_hbm.at[pl.ds(base + cb * _CH, _CH)], wsems[b]
            ).wait()

    return k(node_tokens, tgt3)


def _tc_logits(edge_tokens, tails, st_row, en_row, q, lng_col, lnb_row, W1,
               b1_row, W2, b2_2d):
    def body(et_ref, tl_ref, st_ref, en_ref, q_ref, lng_ref, lnb_ref, b1_ref,
             w1_ref, w2_ref, b2_ref, out_ref):
        f32 = jnp.float32
        e = et_ref[...]
        t = tl_ref[...]
        w1 = w1_ref[...]
        w1g = w1 * lng_ref[...]
        wbar = jnp.sum(w1g, axis=0, keepdims=True)
        hb = jnp.dot(lnb_ref[...], w1, preferred_element_type=f32) + b1_ref[...]
        w1e = w1g[:_HID]
        w1q = w1g[_HID:2 * _HID]
        w1t = w1g[2 * _HID:]
        qt = q_ref[...]
        qp = jnp.dot(qt, w1q, preferred_element_type=f32)
        sq = jnp.sum(qt, axis=1, keepdims=True)
        sqq = jnp.sum(qt * qt, axis=1, keepdims=True)
        eid = pl.program_id(0) * _EB + lax.broadcasted_iota(
            jnp.int32, (_EB, 1), 0
        )
        onehot = ((eid >= st_ref[...]) & (eid < en_ref[...])).astype(f32)
        qpe = jnp.dot(onehot, qp, preferred_element_type=f32)
        sqe = jnp.dot(onehot, sq, preferred_element_type=f32)
        sqqe = jnp.dot(onehot, sqq, preferred_element_type=f32)
        xw = (jnp.dot(e, w1e, preferred_element_type=f32)
              + jnp.dot(t, w1t, preferred_element_type=f32) + qpe)
        ones_col = jnp.ones((_HID, 1), f32)
        u = e + t
        v = e * e + t * t
        suv = jnp.dot(u, ones_col, preferred_element_type=f32)
        svv = jnp.dot(v, ones_col, preferred_element_type=f32)
        mu = (suv + sqe) * (1.0 / 384.0)
        var = (svv + sqqe) * (1.0 / 384.0) - mu * mu
        r = lax.rsqrt(var + 1e-5)
        h = xw * r - (mu * r) * wbar + hb
        gel = 0.5 * h * (1.0 + lax.erf(h * 0.7071067811865476))
        lg = jnp.dot(gel, w2_ref[...], preferred_element_type=f32) \
            + b2_ref[...]
        out_ref[...] = lg

    return pl.pallas_call(
        body,
        grid=(_NB,),
        in_specs=[
            pl.BlockSpec((_EB, _HID), lambda i: (i, 0)),
            pl.BlockSpec((_EB, _HID), lambda i: (i, 0)),
            pl.BlockSpec((1, _NG), lambda i: (0, 0)),
            pl.BlockSpec((1, _NG), lambda i: (0, 0)),
            pl.BlockSpec((_NG, _HID), lambda i: (0, 0)),
            pl.BlockSpec((3 * _HID, 1), lambda i: (0, 0)),
            pl.BlockSpec((1, 3 * _HID), lambda i: (0, 0)),
            pl.BlockSpec((1, _HID), lambda i: (0, 0)),
            pl.BlockSpec((3 * _HID, _HID), lambda i: (0, 0)),
            pl.BlockSpec((_HID, 1), lambda i: (0, 0)),
            pl.BlockSpec((1, 1), lambda i: (0, 0)),
        ],
        out_specs=pl.BlockSpec((_EB, 1), lambda i: (i, 0)),
        out_shape=jax.ShapeDtypeStruct((_N_EDGES, 1), jnp.float32),
    )(edge_tokens, tails, st_row, en_row, q, lng_col, lnb_row, b1_row, W1,
      W2, b2_2d)


_CHS = 80
_NCHS = _EPT // _CHS


def _sc_denom(lg2, tgt3):
    @functools.partial(
        pl.kernel,
        mesh=_sc_mesh(),
        out_type=(
            jax.ShapeDtypeStruct((2, _NPAD), jnp.float32),
            jax.ShapeDtypeStruct((2, 16), jnp.float32),
        ),
        scratch_types=[
            pltpu.VMEM((_EPT,), jnp.float32),
            pltpu.VMEM((_NCHS, _CHS), jnp.int32),
            pltpu.VMEM((16,), jnp.float32),
            pltpu.VMEM((_NPAD,), jnp.float32),
            pltpu.VMEM_SHARED((_NPAD,), jnp.float32),
            pltpu.VMEM_SHARED((256,), jnp.float32),
        ],
    )
    def k(lg_hbm, tgt_hbm, out_hbm, gm_hbm, lg_v, ix_v, gm_v, z_v, acc_sh,
          max_sh):
        wid = _wid()
        core = lax.axis_index("c")
        sid = lax.axis_index("s")
        pltpu.sync_copy(lg_hbm.at[wid], lg_v)
        pltpu.sync_copy(tgt_hbm.at[wid], ix_v)

        gm_v[...] = jnp.full((16,), -1e30, jnp.float32)

        def mb(j, c):
            gm_v[...] = jnp.maximum(gm_v[...], lg_v[pl.ds(j * 16, 16)])
            return c

        lax.fori_loop(0, _EPT // 16, mb, 0)
        pltpu.sync_copy(gm_v, max_sh.at[pl.ds(sid * 16, 16)])

        @pl.when(sid == 0)
        def _():
            zero = jnp.zeros((16,), jnp.float32)

            def zb(i, c):
                z_v[pl.ds(i * 16, 16)] = zero
                return c

            lax.fori_loop(0, _NPAD // 16, zb, 0)
            pltpu.sync_copy(z_v, acc_sh)

        plsc.subcore_barrier()
        pltpu.sync_copy(max_sh, z_v.at[pl.ds(0, 256)])
        mm = z_v[pl.ds(0, 16)]
        for r in range(1, 16):
            mm = jnp.maximum(mm, z_v[pl.ds(r * 16, 16)])
        s = mm[0]
        for i in range(1, 16):
            s = jnp.maximum(s, mm[i])
        m = jnp.full((16,), s, jnp.float32)
        gm_v[...] = m

        def eb(j, c):
            v = lg_v[pl.ds(j * 16, 16)]
            lg_v[pl.ds(j * 16, 16)] = jnp.exp(v - m)
            return c

        lax.fori_loop(0, _EPT // 16, eb, 0)

        def sb(ci, c):
            pltpu.sync_copy(
                lg_v.at[pl.ds(ci * _CHS, _CHS)],
                acc_sh.at[ix_v.at[ci]],
                add=True,
            )
            return c

        lax.fori_loop(0, _NCHS, sb, 0)
        plsc.subcore_barrier()

        @pl.when(sid == 0)
        def _():
            pltpu.sync_copy(acc_sh, out_hbm.at[core])
            pltpu.sync_copy(gm_v, gm_hbm.at[core])

    return k(lg2, tgt3)


def _tc_nodeoff(parts, gmax2):
    def body(p_ref, m_ref, o_ref):
        m0 = jnp.max(m_ref[0:1, :])
        m1 = jnp.max(m_ref[1:2, :])
        M = jnp.maximum(m0, m1)
        den = (p_ref[0:1, :] * jnp.exp(m0 - M)
               + p_ref[1:2, :] * jnp.exp(m1 - M))
        o_ref[...] = M + jnp.log(den)

    return pl.pallas_call(
        body, out_shape=jax.ShapeDtypeStruct((1, _NPAD), jnp.float32)
    )(parts, gmax2)


def _sc_graphsum(nodeoff, tgt3, lg2, eb3, sel3):
    @functools.partial(
        pl.kernel,
        mesh=_sc_mesh(),
        out_type=jax.ShapeDtypeStruct((2, 512), jnp.float32),
        scratch_types=[
            pltpu.VMEM((_NCHS, _CHS), jnp.int32),
            pltpu.VMEM_SHARED((_NPAD,), jnp.float32),
            pltpu.VMEM((_EPT,), jnp.float32),
            pltpu.VMEM((_EPT,), jnp.float32),
            pltpu.VMEM((_NCHS, _CHS), jnp.float32),
            pltpu.VMEM((_NCHS, _CHS), jnp.float32),
            pltpu.VMEM((_NCHS, _CHS), jnp.int32),
            pltpu.VMEM((_NCHS, _CHS), jnp.int32),
            pltpu.VMEM((512,), jnp.float32),
            pltpu.VMEM_SHARED((512,), jnp.float32),
        ],
    )
    def k(off_hbm, tgt_hbm, lg_hbm, eb_hbm, sel_hbm, out_hbm,
          idx_v, off_sh, ofs_v, lg_v, lps_v, sel_v, kv_v, kv2_v, z_v, acc_sh):
        wid = _wid()
        core = lax.axis_index("c")
        sid = lax.axis_index("s")
        pltpu.sync_copy(tgt_hbm.at[wid], idx_v)
        pltpu.sync_copy(lg_hbm.at[wid], lg_v)
        pltpu.sync_copy(eb_hbm.at[wid], kv_v)
        pltpu.sync_copy(sel_hbm.at[wid], sel_v)

        @pl.when(sid == 0)
        def _():
            pltpu.sync_copy(off_hbm, off_sh)
            zero = jnp.zeros((16,), jnp.float32)

            def zb(i, c):
                z_v[pl.ds(i * 16, 16)] = zero
                return c

            lax.fori_loop(0, 512 // 16, zb, 0)
            pltpu.sync_copy(z_v, acc_sh)

        plsc.subcore_barrier()

        def gb(ci, c):
            pltpu.sync_copy(
                off_sh.at[idx_v.at[ci]], ofs_v.at[pl.ds(ci * _CHS, _CHS)]
            )
            return c

        lax.fori_loop(0, _NCHS, gb, 0)

        kbase = sid * 16

        def cb(j, c):
            ci = j // (_CHS // 16)
            l = (j % (_CHS // 16)) * 16
            v = lg_v[pl.ds(j * 16, 16)]
            o = ofs_v[pl.ds(j * 16, 16)]
            s = sel_v[ci, pl.ds(l, 16)]
            g = kv_v[ci, pl.ds(l, 16)]
            lps_v[ci, pl.ds(l, 16)] = (v - o) * s
            kv_v[ci, pl.ds(l, 16)] = g + kbase
            kv2_v[ci, pl.ds(l, 16)] = g + (kbase + 256)
            return c

        lax.fori_loop(0, _EPT // 16, cb, 0)

        def sb(ci, c):
            pltpu.sync_copy(lps_v.at[ci], acc_sh.at[kv_v.at[ci]], add=True)
            pltpu.sync_copy(sel_v.at[ci], acc_sh.at[kv2_v.at[ci]], add=True)
            return c

        lax.fori_loop(0, _NCHS, sb, 0)
        plsc.subcore_barrier()

        @pl.when(sid == 0)
        def _():
            pltpu.sync_copy(acc_sh, out_hbm.at[core])

    return k(nodeoff, tgt3, lg2, eb3, sel3)


def _tc_final(parts):
    def body(p_ref, o1_ref, o2_ref):
        f32 = jnp.float32
        x = p_ref[...]
        tot = (jnp.sum(x[0:16], axis=0, keepdims=True)
               + jnp.sum(x[32:48], axis=0, keepdims=True))
        cnt = (jnp.sum(x[16:32], axis=0, keepdims=True)
               + jnp.sum(x[48:64], axis=0, keepdims=True))
        has = cnt > 0.0
        nll = jnp.where(has, -tot, 0.0)
        ngr = jnp.maximum(jnp.sum(has.astype(f32)), 1.0)
        o1_ref[...] = tot
        o2_ref[...] = (jnp.sum(nll) / ngr).reshape(1, 1)

    return pl.pallas_call(
        body,
        out_shape=[
            jax.ShapeDtypeStruct((1, _NG), jnp.float32),
            jax.ShapeDtypeStruct((1, 1), jnp.float32),
        ],
    )(parts)


def kernel(edge_tokens, node_tokens, question_tokens, ln_g, ln_b, W1, b1, W2,
           b2, edge_batch, selected_mask, edge_index):
    tgt = edge_index[1]
    tails = _sc_gather_tails(node_tokens, tgt.reshape(_NW, _NCH, _CH))
    starts = jnp.searchsorted(
        edge_batch, jnp.arange(_NG, dtype=edge_batch.dtype)
    ).astype(jnp.int32)
    ends = jnp.concatenate(
        [starts[1:], jnp.full((1,), _N_EDGES, jnp.int32)]
    )
    logits = _tc_logits(
        edge_tokens, tails,
        starts.reshape(1, _NG),
        ends.reshape(1, _NG),
        question_tokens,
        ln_g.reshape(3 * _HID, 1),
        ln_b.reshape(1, 3 * _HID),
        W1,
        b1.reshape(1, _HID),
        W2,
        b2.reshape(1, 1),
    )
    lg2 = logits.reshape(_NW, _EPT)
    tgt3 = tgt.reshape(_NW, _NCHS, _CHS)
    denp, gmax2 = _sc_denom(lg2, tgt3)
    nodeoff = _tc_nodeoff(denp, gmax2)
    parts = _sc_graphsum(
        nodeoff.reshape(_NPAD), tgt3, lg2,
        edge_batch.reshape(_NW, _NCHS, _CHS),
        selected_mask.astype(jnp.float32).reshape(_NW, _NCHS, _CHS),
    )
    logpb, nll = _tc_final(parts.reshape(64, _NG))
    return logpb.reshape(_NG), nll.reshape(())
